# baseline (speedup 1.0000x reference)
import jax
import jax.numpy as jnp
from jax.experimental import pallas as pl


def _bn(x):
    mu = jnp.mean(x, axis=0, keepdims=True)
    var = jnp.var(x, axis=0, keepdims=True)
    return (x - mu) / jnp.sqrt(var + 1e-5)


def _conv(x, e, u, src, dst, seg, p, residual=True):
    act = jax.nn.relu
    e_new = act(e @ p['We'] + x[src] @ p['Wsrc'] + x[dst] @ p['Wdst'] + p['be'])
    gate = jax.nn.sigmoid(e_new)
    agg = jax.ops.segment_sum(gate * (x[src] @ p['Wm']), dst, num_segments=x.shape[0])
    x_new = act(x @ p['Wx'] + agg + (u @ p['Wug'])[seg] + p['bx'])
    cnt = jax.ops.segment_sum(jnp.ones((x.shape[0], 1), jnp.float32), seg, num_segments=u.shape[0])
    u_pool = jax.ops.segment_sum(x_new, seg, num_segments=u.shape[0]) / jnp.maximum(cnt, 1.0)
    u_new = act(u @ p['Wu'] + u_pool @ p['Wg'] + p['bu'])
    x_new = _bn(x_new); e_new = _bn(e_new); u_new = _bn(u_new)
    if residual:
        x_new = x_new + x; e_new = e_new + e; u_new = u_new + u
    return x_new, e_new, u_new


def _identity_kernel(h_ref, o_ref):
    o_ref[...] = h_ref[...]


def kernel(atom_feats, bond_feats, global_feats, mol_edge_index, rxn_edge_index, atom2mol, atom2rxn, params):
    R = 250
    x = atom_feats @ params['emb_atom_W'] + params['emb_atom_b']
    e = bond_feats @ params['emb_bond_W'] + params['emb_bond_b']
    u = global_feats @ params['emb_glob_W'] + params['emb_glob_b']
    src, dst = mol_edge_index[0], mol_edge_index[1]
    for p in params['mol_convs']:
        x, e, u = _conv(x, e, u, src, dst, atom2mol, p)
    cnt = jax.ops.segment_sum(jnp.ones((x.shape[0], 1), jnp.float32), atom2rxn, num_segments=R)
    u_rxn = jax.ops.segment_sum(x, atom2rxn, num_segments=R) / jnp.maximum(cnt, 1.0)
    rsrc, rdst = rxn_edge_index[0], rxn_edge_index[1]
    for p in params['rxn_convs']:
        x, e, u_rxn = _conv(x, e, u_rxn, rsrc, rdst, atom2rxn, p)
    h = e
    n = len(params['dec_Ws'])
    for i in range(n):
        h = h @ params['dec_Ws'][i] + params['dec_bs'][i]
        if i < n - 1:
            h = jax.nn.relu(h)
    h = pl.pallas_call(
        _identity_kernel,
        out_shape=jax.ShapeDtypeStruct(h.shape, h.dtype),
        grid=(20,),
        in_specs=[pl.BlockSpec((h.shape[0] // 20, h.shape[1]), lambda i: (i, 0))],
        out_specs=pl.BlockSpec((h.shape[0] // 20, h.shape[1]), lambda i: (i, 0)),
    )(h)
    return h


# speedup vs baseline: 2.2882x; 2.2882x over previous
import functools

import jax
import jax.numpy as jnp
from jax import lax
from jax.experimental import pallas as pl
from jax.experimental.pallas import tpu as pltpu
from jax.experimental.pallas import tpu_sc as plsc

N = 10000
E = 160000
M = 500
R = 250
H = 128
NSEG = 512
NC, NS = 2, 16
NW = NC * NS
EPW = E // NW
CH = 128
NCH = EPW // CH
TAIL = EPW - NCH * CH
EPS = 1e-5

f32 = jnp.float32


def _mesh():
    return plsc.VectorSubcoreMesh(core_axis_name="c", subcore_axis_name="s",
                                  num_cores=NC, num_subcores=NS)


def _sc_gather3(t_src, t_dst, t_msg, src, dst):
    @functools.partial(
        pl.kernel,
        out_type=[jax.ShapeDtypeStruct((E, H), f32)] * 3,
        mesh=_mesh(),
        scratch_types=[
            pltpu.VMEM((CH,), jnp.int32), pltpu.VMEM((CH,), jnp.int32),
            pltpu.VMEM((CH, H), f32), pltpu.VMEM((CH, H), f32), pltpu.VMEM((CH, H), f32),
            pltpu.VMEM((TAIL,), jnp.int32), pltpu.VMEM((TAIL,), jnp.int32),
            pltpu.VMEM((TAIL, H), f32), pltpu.VMEM((TAIL, H), f32), pltpu.VMEM((TAIL, H), f32),
            pltpu.SemaphoreType.DMA,
        ],
    )
    def k(t1h, t2h, t3h, srch, dsth, g1h, g2h, g3h,
          isv, idv, r1, r2, r3, isvT, idvT, r1T, r2T, r3T, sem):
        wid = lax.axis_index("s") * NC + lax.axis_index("c")
        base = wid * EPW

        def do_chunk(off, n, ivs, ivd, b1, b2, b3):
            pltpu.sync_copy(srch.at[pl.ds(off, n)], ivs)
            pltpu.sync_copy(dsth.at[pl.ds(off, n)], ivd)
            a = pltpu.async_copy(t1h.at[ivs], b1, sem)
            b = pltpu.async_copy(t2h.at[ivd], b2, sem)
            c = pltpu.async_copy(t3h.at[ivs], b3, sem)
            a.wait(); b.wait(); c.wait()
            pltpu.sync_copy(b1, g1h.at[pl.ds(off, n)])
            pltpu.sync_copy(b2, g2h.at[pl.ds(off, n)])
            pltpu.sync_copy(b3, g3h.at[pl.ds(off, n)])

        def body(i, carry):
            do_chunk(base + i * CH, CH, isv, idv, r1, r2, r3)
            return carry

        lax.fori_loop(0, NCH, body, 0)
        do_chunk(base + NCH * CH, TAIL, isvT, idvT, r1T, r2T, r3T)

    return k(t_src, t_dst, t_msg, src, dst)


def _sc_gather2(t_src, t_dst, src, dst):
    @functools.partial(
        pl.kernel,
        out_type=[jax.ShapeDtypeStruct((E, H), f32)] * 2,
        mesh=_mesh(),
        scratch_types=[
            pltpu.VMEM((CH,), jnp.int32), pltpu.VMEM((CH,), jnp.int32),
            pltpu.VMEM((CH, H), f32), pltpu.VMEM((CH, H), f32),
            pltpu.VMEM((TAIL,), jnp.int32), pltpu.VMEM((TAIL,), jnp.int32),
            pltpu.VMEM((TAIL, H), f32), pltpu.VMEM((TAIL, H), f32),
            pltpu.SemaphoreType.DMA,
        ],
    )
    def k(t1h, t2h, srch, dsth, g1h, g2h,
          isv, idv, r1, r2, isvT, idvT, r1T, r2T, sem):
        wid = lax.axis_index("s") * NC + lax.axis_index("c")
        base = wid * EPW

        def do_chunk(off, n, ivs, ivd, b1, b2):
            pltpu.sync_copy(srch.at[pl.ds(off, n)], ivs)
            pltpu.sync_copy(dsth.at[pl.ds(off, n)], ivd)
            a = pltpu.async_copy(t1h.at[ivs], b1, sem)
            b = pltpu.async_copy(t2h.at[ivd], b2, sem)
            a.wait(); b.wait()
            pltpu.sync_copy(b1, g1h.at[pl.ds(off, n)])
            pltpu.sync_copy(b2, g2h.at[pl.ds(off, n)])

        def body(i, carry):
            do_chunk(base + i * CH, CH, isv, idv, r1, r2)
            return carry

        lax.fori_loop(0, NCH, body, 0)
        do_chunk(base + NCH * CH, TAIL, isvT, idvT, r1T, r2T)

    return k(t_src, t_dst, src, dst)


def _sc_scatter(msg, dst, zeros_nh):
    @functools.partial(
        pl.kernel,
        out_type=jax.ShapeDtypeStruct((NC * N, H), f32),
        mesh=_mesh(),
        scratch_types=[
            pltpu.VMEM_SHARED((N, H), f32),
            pltpu.VMEM((CH,), jnp.int32), pltpu.VMEM((CH, H), f32),
            pltpu.VMEM((TAIL,), jnp.int32), pltpu.VMEM((TAIL, H), f32),
        ],
    )
    def k(msgh, dsth, zh, outh, acc, idxv, rows, idxT, rowsT):
        cid = lax.axis_index("c")
        sid = lax.axis_index("s")
        wid = sid * NC + cid
        base = wid * EPW

        @pl.when(sid < NS - 1)
        def _():
            pltpu.sync_copy(zh.at[pl.ds(sid * 640, 640)], acc.at[pl.ds(sid * 640, 640)])

        @pl.when(sid == NS - 1)
        def _():
            pltpu.sync_copy(zh.at[pl.ds(9600, 400)], acc.at[pl.ds(9600, 400)])

        plsc.subcore_barrier()

        def do_chunk(off, n, iv, rv):
            pltpu.sync_copy(msgh.at[pl.ds(off, n)], rv)
            pltpu.sync_copy(dsth.at[pl.ds(off, n)], iv)
            pltpu.sync_copy(rv, acc.at[iv], add=True)

        def body(i, carry):
            do_chunk(base + i * CH, CH, idxv, rows)
            return carry

        lax.fori_loop(0, NCH, body, 0)
        do_chunk(base + NCH * CH, TAIL, idxT, rowsT)

        plsc.subcore_barrier()

        @pl.when(sid < NS - 1)
        def _():
            pltpu.sync_copy(acc.at[pl.ds(sid * 640, 640)],
                            outh.at[pl.ds(cid * N + sid * 640, 640)])

        @pl.when(sid == NS - 1)
        def _():
            pltpu.sync_copy(acc.at[pl.ds(9600, 400)],
                            outh.at[pl.ds(cid * N + 9600, 400)])

    return k(msg, dst, zeros_nh)


def _dot(a, b):
    return jnp.dot(a, b, preferred_element_type=f32)


def _embed_xu_body(af, wa, ba, gf, wg, bg, xo, uo):
    xo[...] = _dot(af[...], wa[...]) + ba[...]
    uo[...] = _dot(gf[...], wg[...]) + bg[...]


def _embed_xu(atom_feats, wa, ba, gf_pad, wg, bg):
    return pl.pallas_call(
        _embed_xu_body,
        out_shape=[jax.ShapeDtypeStruct((N, H), f32),
                   jax.ShapeDtypeStruct((NSEG, H), f32)],
    )(atom_feats, wa, ba.reshape(1, H), gf_pad, wg, bg.reshape(1, H))


def _embed_e_body(bf, w, b, o):
    o[...] = _dot(bf[...], w[...]) + b[...]


def _embed_e(bond_feats, w, b):
    BE = 8000
    return pl.pallas_call(
        _embed_e_body,
        grid=(E // BE,),
        in_specs=[pl.BlockSpec((BE, 64), lambda i: (i, 0)),
                  pl.BlockSpec((64, H), lambda i: (0, 0)),
                  pl.BlockSpec((1, H), lambda i: (0, 0))],
        out_specs=pl.BlockSpec((BE, H), lambda i: (i, 0)),
        out_shape=jax.ShapeDtypeStruct((E, H), f32),
    )(bond_feats, w, b.reshape(1, H))


def _proj4_body(x, ws, wd, wm, u, wug, gso, gdo, xmo, ugo):
    xv = x[...]
    gso[...] = _dot(xv, ws[...])
    gdo[...] = _dot(xv, wd[...])
    xmo[...] = _dot(xv, wm[...])
    ugo[...] = _dot(u[...], wug[...])


def _proj4(x, u, p):
    return pl.pallas_call(
        _proj4_body,
        out_shape=[jax.ShapeDtypeStruct((N, H), f32)] * 3
        + [jax.ShapeDtypeStruct((NSEG, H), f32)],
    )(x, p['Wsrc'], p['Wdst'], p['Wm'], u, p['Wug'])


def _proj2_body(x, ws, wd, gso, gdo):
    xv = x[...]
    gso[...] = _dot(xv, ws[...])
    gdo[...] = _dot(xv, wd[...])


def _proj2(x, p):
    return pl.pallas_call(
        _proj2_body,
        out_shape=[jax.ShapeDtypeStruct((N, H), f32)] * 2,
    )(x, p['Wsrc'], p['Wdst'])


def _epass_body(e, g1, g2, xms, we, be, epre_o, msg_o, s_o, ss_o):
    i = pl.program_id(0)
    epre = jnp.maximum(_dot(e[...], we[...]) + g1[...] + g2[...] + be[...], 0.0)
    epre_o[...] = epre
    msg_o[...] = jax.nn.sigmoid(epre) * xms[...]
    s = jnp.sum(epre, axis=0, keepdims=True)
    ss = jnp.sum(epre * epre, axis=0, keepdims=True)

    @pl.when(i == 0)
    def _():
        s_o[...] = s
        ss_o[...] = ss

    @pl.when(i != 0)
    def _():
        s_o[...] += s
        ss_o[...] += ss


def _epass(e, g1, g2, xms, p):
    BE = 4000
    blk = lambda i: (i, 0)
    zero = lambda i: (0, 0)
    return pl.pallas_call(
        _epass_body,
        grid=(E // BE,),
        in_specs=[pl.BlockSpec((BE, H), blk)] * 4
        + [pl.BlockSpec((H, H), zero), pl.BlockSpec((1, H), zero)],
        out_specs=[pl.BlockSpec((BE, H), blk), pl.BlockSpec((BE, H), blk),
                   pl.BlockSpec((1, H), zero), pl.BlockSpec((1, H), zero)],
        out_shape=[jax.ShapeDtypeStruct((E, H), f32), jax.ShapeDtypeStruct((E, H), f32),
                   jax.ShapeDtypeStruct((1, H), f32), jax.ShapeDtypeStruct((1, H), f32)],
    )(e, g1, g2, xms, p['We'], p['be'].reshape(1, H))


def _epass_last_body(e, g1, g2, we, be, epre_o, s_o, ss_o):
    i = pl.program_id(0)
    epre = jnp.maximum(_dot(e[...], we[...]) + g1[...] + g2[...] + be[...], 0.0)
    epre_o[...] = epre
    s = jnp.sum(epre, axis=0, keepdims=True)
    ss = jnp.sum(epre * epre, axis=0, keepdims=True)

    @pl.when(i == 0)
    def _():
        s_o[...] = s
        ss_o[...] = ss

    @pl.when(i != 0)
    def _():
        s_o[...] += s
        ss_o[...] += ss


def _epass_last(e, g1, g2, p):
    BE = 4000
    blk = lambda i: (i, 0)
    zero = lambda i: (0, 0)
    return pl.pallas_call(
        _epass_last_body,
        grid=(E // BE,),
        in_specs=[pl.BlockSpec((BE, H), blk)] * 3
        + [pl.BlockSpec((H, H), zero), pl.BlockSpec((1, H), zero)],
        out_specs=[pl.BlockSpec((BE, H), blk),
                   pl.BlockSpec((1, H), zero), pl.BlockSpec((1, H), zero)],
        out_shape=[jax.ShapeDtypeStruct((E, H), f32),
                   jax.ShapeDtypeStruct((1, H), f32), jax.ShapeDtypeStruct((1, H), f32)],
    )(e, g1, g2, p['We'], p['be'].reshape(1, H))


def _xpass_body(bx_, x, wx, a0, a1, ug, seg, xpre_o, s_o, ss_o, pool_o):
    i = pl.program_id(0)
    segv = seg[...]
    oh = (segv == lax.broadcasted_iota(jnp.int32, (bx_, NSEG), 1)).astype(f32)
    ugs = _dot(oh, ug[...])
    xpre = jnp.maximum(_dot(x[...], wx[...]) + a0[...] + a1[...] + ugs, 0.0)
    xpre_o[...] = xpre
    x2 = jnp.concatenate([xpre, jnp.ones((bx_, H), f32)], axis=1)
    pool = lax.dot_general(oh, x2, (((0,), (0,)), ((), ())),
                           preferred_element_type=f32)
    s = jnp.sum(xpre, axis=0, keepdims=True)
    ss = jnp.sum(xpre * xpre, axis=0, keepdims=True)

    @pl.when(i == 0)
    def _():
        s_o[...] = s
        ss_o[...] = ss
        pool_o[...] = pool

    @pl.when(i != 0)
    def _():
        s_o[...] += s
        ss_o[...] += ss
        pool_o[...] += pool


def _xpass_call(x, wx, a0, a1, ug, seg2d):
    B = 2000
    blk = lambda i: (i, 0)
    zero = lambda i: (0, 0)
    return pl.pallas_call(
        functools.partial(_xpass_body, B),
        grid=(N // B,),
        in_specs=[pl.BlockSpec((B, H), blk), pl.BlockSpec((H, H), zero),
                  pl.BlockSpec((B, H), blk), pl.BlockSpec((B, H), blk),
                  pl.BlockSpec((NSEG, H), zero), pl.BlockSpec((B, 1), blk)],
        out_specs=[pl.BlockSpec((B, H), blk),
                   pl.BlockSpec((1, H), zero), pl.BlockSpec((1, H), zero),
                   pl.BlockSpec((NSEG, 2 * H), zero)],
        out_shape=[jax.ShapeDtypeStruct((N, H), f32),
                   jax.ShapeDtypeStruct((1, H), f32), jax.ShapeDtypeStruct((1, H), f32),
                   jax.ShapeDtypeStruct((NSEG, 2 * H), f32)],
    )(x, wx, a0, a1, ug, seg2d)


def _xfin_body(xpre, x, s, ss, o):
    mu = s[...] / N
    var = ss[...] / N - mu * mu
    o[...] = (xpre[...] - mu) * lax.rsqrt(var + EPS) + x[...]


def _xfin(xpre, x, s, ss):
    return pl.pallas_call(
        _xfin_body,
        out_shape=jax.ShapeDtypeStruct((N, H), f32),
    )(xpre, x, s, ss)


def _uupdate_body(u, wu, wg, bu, pool, uo):
    u_pool = pool[:, :H] / jnp.maximum(pool[:, H:], 1.0)
    upre = jnp.maximum(_dot(u[...], wu[...]) + _dot(u_pool, wg[...]) + bu[...], 0.0)
    mask = (lax.broadcasted_iota(jnp.int32, (NSEG, H), 0) < M).astype(f32)
    mu = jnp.sum(upre * mask, axis=0, keepdims=True) / M
    var = jnp.sum(((upre - mu) * mask) ** 2, axis=0, keepdims=True) / M
    uo[...] = (upre - mu) * lax.rsqrt(var + EPS) + u[...]


def _uupdate(u, pool, p):
    return pl.pallas_call(
        _uupdate_body,
        out_shape=jax.ShapeDtypeStruct((NSEG, H), f32),
    )(u, p['Wu'], p['Wg'], p['bu'].reshape(1, H), pool)


def _upool_body(x, seg, uo):
    segv = seg[...]
    oh = (segv == lax.broadcasted_iota(jnp.int32, (N, NSEG), 1)).astype(f32)
    x2 = jnp.concatenate([x[...], jnp.ones((N, H), f32)], axis=1)
    pool = lax.dot_general(oh, x2, (((0,), (0,)), ((), ())),
                           preferred_element_type=f32)
    uo[...] = pool[:, :H] / jnp.maximum(pool[:, H:], 1.0)


def _upool(x, seg2d):
    return pl.pallas_call(
        _upool_body,
        out_shape=jax.ShapeDtypeStruct((NSEG, H), f32),
    )(x, seg2d)


def _efin_body(epre, e, s, ss, o):
    mu = s[...] / E
    var = ss[...] / E - mu * mu
    o[...] = (epre[...] - mu) * lax.rsqrt(var + EPS) + e[...]


def _efin(epre, e, s, ss):
    BE = 8000
    blk = lambda i: (i, 0)
    zero = lambda i: (0, 0)
    return pl.pallas_call(
        _efin_body,
        grid=(E // BE,),
        in_specs=[pl.BlockSpec((BE, H), blk), pl.BlockSpec((BE, H), blk),
                  pl.BlockSpec((1, H), zero), pl.BlockSpec((1, H), zero)],
        out_specs=pl.BlockSpec((BE, H), blk),
        out_shape=jax.ShapeDtypeStruct((E, H), f32),
    )(epre, e, s, ss)


def _efin_dec_body(epre, e, s, ss, w1, b1, w2, b2, w3, b3, o):
    mu = s[...] / E
    var = ss[...] / E - mu * mu
    h = (epre[...] - mu) * lax.rsqrt(var + EPS) + e[...]
    h = jnp.maximum(_dot(h, w1[...]) + b1[...], 0.0)
    h = jnp.maximum(_dot(h, w2[...]) + b2[...], 0.0)
    o[...] = _dot(h, w3[...]) + b3[...]


def _efin_dec(epre, e, s, ss, dec_ws, dec_bs):
    BE = 8000
    blk = lambda i: (i, 0)
    zero = lambda i: (0, 0)
    w1, w2, w3 = dec_ws
    b1, b2, b3 = dec_bs
    return pl.pallas_call(
        _efin_dec_body,
        grid=(E // BE,),
        in_specs=[pl.BlockSpec((BE, H), blk), pl.BlockSpec((BE, H), blk),
                  pl.BlockSpec((1, H), zero), pl.BlockSpec((1, H), zero),
                  pl.BlockSpec((H, H), zero), pl.BlockSpec((1, H), zero),
                  pl.BlockSpec((H, 64), zero), pl.BlockSpec((1, 64), zero),
                  pl.BlockSpec((64, 3), zero), pl.BlockSpec((1, 3), zero)],
        out_specs=pl.BlockSpec((BE, 3), blk),
        out_shape=jax.ShapeDtypeStruct((E, 3), f32),
    )(epre, e, s, ss, w1, b1.reshape(1, H), w2, b2.reshape(1, 64), w3,
      b3.reshape(1, 3))


def kernel(atom_feats, bond_feats, global_feats, mol_edge_index, rxn_edge_index,
           atom2mol, atom2rxn, params):
    p = params
    srcm = mol_edge_index[0].astype(jnp.int32)
    dstm = mol_edge_index[1].astype(jnp.int32)
    srcr = rxn_edge_index[0].astype(jnp.int32)
    dstr = rxn_edge_index[1].astype(jnp.int32)
    seg_mol = atom2mol.astype(jnp.int32).reshape(N, 1)
    seg_rxn = atom2rxn.astype(jnp.int32).reshape(N, 1)
    zeros_nh = jnp.zeros((N, H), f32)

    gf_pad = jnp.zeros((NSEG, global_feats.shape[1]), f32).at[:M].set(global_feats)

    x, u = _embed_xu(atom_feats, p['emb_atom_W'], p['emb_atom_b'],
                     gf_pad, p['emb_glob_W'], p['emb_glob_b'])
    e = _embed_e(bond_feats, p['emb_bond_W'], p['emb_bond_b'])

    def conv_layer(x, e, u, src, dst, seg2d, cp, need_u):
        gs_t, gd_t, xm_t, ug = _proj4(x, u, cp)
        ug = ug + cp['bx'].reshape(1, H)
        g1, g2, xms = _sc_gather3(gs_t, gd_t, xm_t, src, dst)
        epre, msg, es, ess = _epass(e, g1, g2, xms, cp)
        aggs = _sc_scatter(msg, dst, zeros_nh)
        a0, a1 = aggs[:N], aggs[N:]
        xpre, xs, xss, pool = _xpass_call(x, cp['Wx'], a0, a1, ug, seg2d)
        x_new = _xfin(xpre, x, xs, xss)
        e_new = _efin(epre, e, es, ess)
        u_new = _uupdate(u, pool, cp) if need_u else None
        return x_new, e_new, u_new

    x, e, u = conv_layer(x, e, u, srcm, dstm, seg_mol, p['mol_convs'][0], True)
    x, e, _ = conv_layer(x, e, u, srcm, dstm, seg_mol, p['mol_convs'][1], False)

    u_rxn = _upool(x, seg_rxn)

    x, e, _ = conv_layer(x, e, u_rxn, srcr, dstr, seg_rxn, p['rxn_convs'][0], False)

    cp = p['rxn_convs'][1]
    gs_t, gd_t = _proj2(x, cp)
    g1, g2 = _sc_gather2(gs_t, gd_t, srcr, dstr)
    epre, es, ess = _epass_last(e, g1, g2, cp)
    h = _efin_dec(epre, e, es, ess, p['dec_Ws'], p['dec_bs'])
    return h


# speedup vs baseline: 2.6047x; 1.1383x over previous
import functools

import jax
import jax.numpy as jnp
from jax import lax
from jax.experimental import pallas as pl
from jax.experimental.pallas import tpu as pltpu
from jax.experimental.pallas import tpu_sc as plsc

N = 10000
E = 160000
M = 500
R = 250
H = 128
NSEG = 512
NC, NS = 2, 16
NW = NC * NS
EPW = E // NW
CH = 128
NCH = EPW // CH
TAIL = EPW - NCH * CH
EPS = 1e-5

f32 = jnp.float32


def _mesh():
    return plsc.VectorSubcoreMesh(core_axis_name="c", subcore_axis_name="s",
                                  num_cores=NC, num_subcores=NS)


def _gather_pipelined(tables, idx_sel, n_out):
    ntab = len(tables)

    def body(srch, dsth, outs, isv, idv, bufs, bufsT, semg0, semg1, semw0, semw1):
        wid = lax.axis_index("s") * NC + lax.axis_index("c")
        base = wid * EPW
        pltpu.sync_copy(srch.at[pl.ds(base, EPW)], isv)
        pltpu.sync_copy(dsth.at[pl.ds(base, EPW)], idv)
        ivs = [isv, idv]

        def gstart(i, slot, semg):
            off = i * CH
            for t in range(ntab):
                pltpu.async_copy(
                    tables[t].at[ivs[idx_sel[t]].at[pl.ds(off, CH)]],
                    bufs[t].at[slot], semg)

        def gwait(slot, semg):
            for t in range(ntab):
                pltpu.make_async_copy(
                    tables[t].at[isv.at[pl.ds(0, CH)]], bufs[t].at[slot], semg
                ).wait()

        def wstart(i, slot, semw):
            off = base + i * CH
            for t in range(ntab):
                pltpu.async_copy(bufs[t].at[slot], outs[t].at[pl.ds(off, CH)], semw)

        def wwait(slot, semw):
            for t in range(ntab):
                pltpu.make_async_copy(
                    bufs[t].at[slot], outs[t].at[pl.ds(0, CH)], semw).wait()

        gstart(0, 0, semg0)
        gstart(1, 1, semg1)

        def step(i, carry):
            def for_slot(slot, semg, semw):
                gwait(slot, semg)
                wstart(i, slot, semw)
                wwait(slot, semw)

                @pl.when(i + 2 < NCH)
                def _():
                    gstart(i + 2, slot, semg)

            @pl.when(lax.rem(i, 2) == 0)
            def _():
                for_slot(0, semg0, semw0)

            @pl.when(lax.rem(i, 2) == 1)
            def _():
                for_slot(1, semg1, semw1)

            return carry

        lax.fori_loop(0, NCH, step, 0)

        offT = NCH * CH
        for t in range(ntab):
            pltpu.async_copy(
                tables[t].at[ivs[idx_sel[t]].at[pl.ds(offT, TAIL)]],
                bufsT[t], semg0)
        for t in range(ntab):
            pltpu.make_async_copy(
                tables[t].at[isv.at[pl.ds(0, TAIL)]], bufsT[t], semg0).wait()
        for t in range(ntab):
            pltpu.sync_copy(bufsT[t], outs[t].at[pl.ds(base + offT, TAIL)])

    return body


def _sc_gather3(t_src, t_dst, t_msg, src, dst):
    @functools.partial(
        pl.kernel,
        out_type=[jax.ShapeDtypeStruct((E, H), f32)] * 3,
        mesh=_mesh(),
        scratch_types=[
            pltpu.VMEM((EPW,), jnp.int32), pltpu.VMEM((EPW,), jnp.int32),
            pltpu.VMEM((2, CH, H), f32), pltpu.VMEM((2, CH, H), f32),
            pltpu.VMEM((2, CH, H), f32),
            pltpu.VMEM((TAIL, H), f32), pltpu.VMEM((TAIL, H), f32),
            pltpu.VMEM((TAIL, H), f32),
            pltpu.SemaphoreType.DMA, pltpu.SemaphoreType.DMA,
            pltpu.SemaphoreType.DMA, pltpu.SemaphoreType.DMA,
        ],
    )
    def k(t1h, t2h, t3h, srch, dsth, g1h, g2h, g3h,
          isv, idv, b1, b2, b3, b1T, b2T, b3T, sg0, sg1, sw0, sw1):
        _gather_pipelined([t1h, t2h, t3h], [0, 1, 0], 3)(
            srch, dsth, [g1h, g2h, g3h], isv, idv,
            [b1, b2, b3], [b1T, b2T, b3T], sg0, sg1, sw0, sw1)

    return k(t_src, t_dst, t_msg, src, dst)


def _sc_gather2(t_src, t_dst, src, dst):
    @functools.partial(
        pl.kernel,
        out_type=[jax.ShapeDtypeStruct((E, H), f32)] * 2,
        mesh=_mesh(),
        scratch_types=[
            pltpu.VMEM((EPW,), jnp.int32), pltpu.VMEM((EPW,), jnp.int32),
            pltpu.VMEM((2, CH, H), f32), pltpu.VMEM((2, CH, H), f32),
            pltpu.VMEM((TAIL, H), f32), pltpu.VMEM((TAIL, H), f32),
            pltpu.SemaphoreType.DMA, pltpu.SemaphoreType.DMA,
            pltpu.SemaphoreType.DMA, pltpu.SemaphoreType.DMA,
        ],
    )
    def k(t1h, t2h, srch, dsth, g1h, g2h,
          isv, idv, b1, b2, b1T, b2T, sg0, sg1, sw0, sw1):
        _gather_pipelined([t1h, t2h], [0, 1], 2)(
            srch, dsth, [g1h, g2h], isv, idv,
            [b1, b2], [b1T, b2T], sg0, sg1, sw0, sw1)

    return k(t_src, t_dst, src, dst)


def _sc_scatter(msg, dst, zeros_nh):
    @functools.partial(
        pl.kernel,
        out_type=jax.ShapeDtypeStruct((NC * N, H), f32),
        mesh=_mesh(),
        scratch_types=[
            pltpu.VMEM_SHARED((N, H), f32),
            pltpu.VMEM((CH,), jnp.int32), pltpu.VMEM((CH, H), f32),
            pltpu.VMEM((CH,), jnp.int32), pltpu.VMEM((CH, H), f32),
            pltpu.VMEM((TAIL,), jnp.int32), pltpu.VMEM((TAIL, H), f32),
            pltpu.SemaphoreType.DMA, pltpu.SemaphoreType.DMA,
        ],
    )
    def k(msgh, dsth, zh, outh, acc, i0, r0, i1, r1, idxT, rowsT, sm0, sm1):
        cid = lax.axis_index("c")
        sid = lax.axis_index("s")
        wid = sid * NC + cid
        base = wid * EPW

        @pl.when(sid < NS - 1)
        def _():
            pltpu.sync_copy(zh.at[pl.ds(sid * 640, 640)], acc.at[pl.ds(sid * 640, 640)])

        @pl.when(sid == NS - 1)
        def _():
            pltpu.sync_copy(zh.at[pl.ds(9600, 400)], acc.at[pl.ds(9600, 400)])

        plsc.subcore_barrier()

        def lstart(i, iv, rv, sem):
            off = base + i * CH
            pltpu.async_copy(msgh.at[pl.ds(off, CH)], rv, sem)
            pltpu.async_copy(dsth.at[pl.ds(off, CH)], iv, sem)

        def lwait(iv, rv, sem):
            pltpu.make_async_copy(msgh.at[pl.ds(0, CH)], rv, sem).wait()
            pltpu.make_async_copy(dsth.at[pl.ds(0, CH)], iv, sem).wait()

        lstart(0, i0, r0, sm0)

        def step(i, carry):
            def for_slot(iv, rv, sem, iv2, rv2, sem2):
                lwait(iv, rv, sem)

                @pl.when(i + 1 < NCH)
                def _():
                    lstart(i + 1, iv2, rv2, sem2)

                pltpu.sync_copy(rv, acc.at[iv], add=True)

            @pl.when(lax.rem(i, 2) == 0)
            def _():
                for_slot(i0, r0, sm0, i1, r1, sm1)

            @pl.when(lax.rem(i, 2) == 1)
            def _():
                for_slot(i1, r1, sm1, i0, r0, sm0)

            return carry

        lax.fori_loop(0, NCH, step, 0)

        offT = base + NCH * CH
        pltpu.sync_copy(msgh.at[pl.ds(offT, TAIL)], rowsT)
        pltpu.sync_copy(dsth.at[pl.ds(offT, TAIL)], idxT)
        pltpu.sync_copy(rowsT, acc.at[idxT], add=True)

        plsc.subcore_barrier()

        @pl.when(sid < NS - 1)
        def _():
            pltpu.sync_copy(acc.at[pl.ds(sid * 640, 640)],
                            outh.at[pl.ds(cid * N + sid * 640, 640)])

        @pl.when(sid == NS - 1)
        def _():
            pltpu.sync_copy(acc.at[pl.ds(9600, 400)],
                            outh.at[pl.ds(cid * N + 9600, 400)])

    return k(msg, dst, zeros_nh)


def _dot(a, b):
    return jnp.dot(a, b, preferred_element_type=f32)


def _embed_xu_body(af, wa, ba, gf, wg, bg, xo, uo):
    xo[...] = _dot(af[...], wa[...]) + ba[...]
    uo[...] = _dot(gf[...], wg[...]) + bg[...]


def _embed_xu(atom_feats, wa, ba, gf_pad, wg, bg):
    return pl.pallas_call(
        _embed_xu_body,
        out_shape=[jax.ShapeDtypeStruct((N, H), f32),
                   jax.ShapeDtypeStruct((NSEG, H), f32)],
    )(atom_feats, wa, ba.reshape(1, H), gf_pad, wg, bg.reshape(1, H))


def _embed_e_body(bf, w, b, o):
    o[...] = _dot(bf[...], w[...]) + b[...]


def _embed_e(bond_feats, w, b):
    BE = 8000
    return pl.pallas_call(
        _embed_e_body,
        grid=(E // BE,),
        in_specs=[pl.BlockSpec((BE, 64), lambda i: (i, 0)),
                  pl.BlockSpec((64, H), lambda i: (0, 0)),
                  pl.BlockSpec((1, H), lambda i: (0, 0))],
        out_specs=pl.BlockSpec((BE, H), lambda i: (i, 0)),
        out_shape=jax.ShapeDtypeStruct((E, H), f32),
    )(bond_feats, w, b.reshape(1, H))


def _proj4_body(x, ws, wd, wm, u, wug, gso, gdo, xmo, ugo):
    xv = x[...]
    gso[...] = _dot(xv, ws[...])
    gdo[...] = _dot(xv, wd[...])
    xmo[...] = _dot(xv, wm[...])
    ugo[...] = _dot(u[...], wug[...])


def _proj4(x, u, p):
    return pl.pallas_call(
        _proj4_body,
        out_shape=[jax.ShapeDtypeStruct((N, H), f32)] * 3
        + [jax.ShapeDtypeStruct((NSEG, H), f32)],
    )(x, p['Wsrc'], p['Wdst'], p['Wm'], u, p['Wug'])


def _proj2_body(x, ws, wd, gso, gdo):
    xv = x[...]
    gso[...] = _dot(xv, ws[...])
    gdo[...] = _dot(xv, wd[...])


def _proj2(x, p):
    return pl.pallas_call(
        _proj2_body,
        out_shape=[jax.ShapeDtypeStruct((N, H), f32)] * 2,
    )(x, p['Wsrc'], p['Wdst'])


def _epass_body(e, g1, g2, xms, we, be, epre_o, msg_o, s_o, ss_o):
    i = pl.program_id(0)
    epre = jnp.maximum(_dot(e[...], we[...]) + g1[...] + g2[...] + be[...], 0.0)
    epre_o[...] = epre
    msg_o[...] = jax.nn.sigmoid(epre) * xms[...]
    s = jnp.sum(epre, axis=0, keepdims=True)
    ss = jnp.sum(epre * epre, axis=0, keepdims=True)

    @pl.when(i == 0)
    def _():
        s_o[...] = s
        ss_o[...] = ss

    @pl.when(i != 0)
    def _():
        s_o[...] += s
        ss_o[...] += ss


def _epass(e, g1, g2, xms, p):
    BE = 4000
    blk = lambda i: (i, 0)
    zero = lambda i: (0, 0)
    return pl.pallas_call(
        _epass_body,
        grid=(E // BE,),
        in_specs=[pl.BlockSpec((BE, H), blk)] * 4
        + [pl.BlockSpec((H, H), zero), pl.BlockSpec((1, H), zero)],
        out_specs=[pl.BlockSpec((BE, H), blk), pl.BlockSpec((BE, H), blk),
                   pl.BlockSpec((1, H), zero), pl.BlockSpec((1, H), zero)],
        out_shape=[jax.ShapeDtypeStruct((E, H), f32), jax.ShapeDtypeStruct((E, H), f32),
                   jax.ShapeDtypeStruct((1, H), f32), jax.ShapeDtypeStruct((1, H), f32)],
    )(e, g1, g2, xms, p['We'], p['be'].reshape(1, H))


def _epass_last_body(e, g1, g2, we, be, epre_o, s_o, ss_o):
    i = pl.program_id(0)
    epre = jnp.maximum(_dot(e[...], we[...]) + g1[...] + g2[...] + be[...], 0.0)
    epre_o[...] = epre
    s = jnp.sum(epre, axis=0, keepdims=True)
    ss = jnp.sum(epre * epre, axis=0, keepdims=True)

    @pl.when(i == 0)
    def _():
        s_o[...] = s
        ss_o[...] = ss

    @pl.when(i != 0)
    def _():
        s_o[...] += s
        ss_o[...] += ss


def _epass_last(e, g1, g2, p):
    BE = 4000
    blk = lambda i: (i, 0)
    zero = lambda i: (0, 0)
    return pl.pallas_call(
        _epass_last_body,
        grid=(E // BE,),
        in_specs=[pl.BlockSpec((BE, H), blk)] * 3
        + [pl.BlockSpec((H, H), zero), pl.BlockSpec((1, H), zero)],
        out_specs=[pl.BlockSpec((BE, H), blk),
                   pl.BlockSpec((1, H), zero), pl.BlockSpec((1, H), zero)],
        out_shape=[jax.ShapeDtypeStruct((E, H), f32),
                   jax.ShapeDtypeStruct((1, H), f32), jax.ShapeDtypeStruct((1, H), f32)],
    )(e, g1, g2, p['We'], p['be'].reshape(1, H))


def _xpass_body(bx_, x, wx, a0, a1, ug, seg, xpre_o, s_o, ss_o, pool_o):
    i = pl.program_id(0)
    segv = seg[...]
    oh = (segv == lax.broadcasted_iota(jnp.int32, (bx_, NSEG), 1)).astype(f32)
    ugs = _dot(oh, ug[...])
    xpre = jnp.maximum(_dot(x[...], wx[...]) + a0[...] + a1[...] + ugs, 0.0)
    xpre_o[...] = xpre
    x2 = jnp.concatenate([xpre, jnp.ones((bx_, H), f32)], axis=1)
    pool = lax.dot_general(oh, x2, (((0,), (0,)), ((), ())),
                           preferred_element_type=f32)
    s = jnp.sum(xpre, axis=0, keepdims=True)
    ss = jnp.sum(xpre * xpre, axis=0, keepdims=True)

    @pl.when(i == 0)
    def _():
        s_o[...] = s
        ss_o[...] = ss
        pool_o[...] = pool

    @pl.when(i != 0)
    def _():
        s_o[...] += s
        ss_o[...] += ss
        pool_o[...] += pool


def _xpass_call(x, wx, a0, a1, ug, seg2d):
    B = 2000
    blk = lambda i: (i, 0)
    zero = lambda i: (0, 0)
    return pl.pallas_call(
        functools.partial(_xpass_body, B),
        grid=(N // B,),
        in_specs=[pl.BlockSpec((B, H), blk), pl.BlockSpec((H, H), zero),
                  pl.BlockSpec((B, H), blk), pl.BlockSpec((B, H), blk),
                  pl.BlockSpec((NSEG, H), zero), pl.BlockSpec((B, 1), blk)],
        out_specs=[pl.BlockSpec((B, H), blk),
                   pl.BlockSpec((1, H), zero), pl.BlockSpec((1, H), zero),
                   pl.BlockSpec((NSEG, 2 * H), zero)],
        out_shape=[jax.ShapeDtypeStruct((N, H), f32),
                   jax.ShapeDtypeStruct((1, H), f32), jax.ShapeDtypeStruct((1, H), f32),
                   jax.ShapeDtypeStruct((NSEG, 2 * H), f32)],
    )(x, wx, a0, a1, ug, seg2d)


def _xfin_body(xpre, x, s, ss, o):
    mu = s[...] / N
    var = ss[...] / N - mu * mu
    o[...] = (xpre[...] - mu) * lax.rsqrt(var + EPS) + x[...]


def _xfin(xpre, x, s, ss):
    return pl.pallas_call(
        _xfin_body,
        out_shape=jax.ShapeDtypeStruct((N, H), f32),
    )(xpre, x, s, ss)


def _uupdate_body(u, wu, wg, bu, pool, uo):
    u_pool = pool[:, :H] / jnp.maximum(pool[:, H:], 1.0)
    upre = jnp.maximum(_dot(u[...], wu[...]) + _dot(u_pool, wg[...]) + bu[...], 0.0)
    mask = (lax.broadcasted_iota(jnp.int32, (NSEG, H), 0) < M).astype(f32)
    mu = jnp.sum(upre * mask, axis=0, keepdims=True) / M
    var = jnp.sum(((upre - mu) * mask) ** 2, axis=0, keepdims=True) / M
    uo[...] = (upre - mu) * lax.rsqrt(var + EPS) + u[...]


def _uupdate(u, pool, p):
    return pl.pallas_call(
        _uupdate_body,
        out_shape=jax.ShapeDtypeStruct((NSEG, H), f32),
    )(u, p['Wu'], p['Wg'], p['bu'].reshape(1, H), pool)


def _upool_body(x, seg, uo):
    segv = seg[...]
    oh = (segv == lax.broadcasted_iota(jnp.int32, (N, NSEG), 1)).astype(f32)
    x2 = jnp.concatenate([x[...], jnp.ones((N, H), f32)], axis=1)
    pool = lax.dot_general(oh, x2, (((0,), (0,)), ((), ())),
                           preferred_element_type=f32)
    uo[...] = pool[:, :H] / jnp.maximum(pool[:, H:], 1.0)


def _upool(x, seg2d):
    return pl.pallas_call(
        _upool_body,
        out_shape=jax.ShapeDtypeStruct((NSEG, H), f32),
    )(x, seg2d)


def _efin_body(epre, e, s, ss, o):
    mu = s[...] / E
    var = ss[...] / E - mu * mu
    o[...] = (epre[...] - mu) * lax.rsqrt(var + EPS) + e[...]


def _efin(epre, e, s, ss):
    BE = 8000
    blk = lambda i: (i, 0)
    zero = lambda i: (0, 0)
    return pl.pallas_call(
        _efin_body,
        grid=(E // BE,),
        in_specs=[pl.BlockSpec((BE, H), blk), pl.BlockSpec((BE, H), blk),
                  pl.BlockSpec((1, H), zero), pl.BlockSpec((1, H), zero)],
        out_specs=pl.BlockSpec((BE, H), blk),
        out_shape=jax.ShapeDtypeStruct((E, H), f32),
    )(epre, e, s, ss)


def _efin_dec_body(epre, e, s, ss, w1, b1, w2, b2, w3, b3, o):
    mu = s[...] / E
    var = ss[...] / E - mu * mu
    h = (epre[...] - mu) * lax.rsqrt(var + EPS) + e[...]
    h = jnp.maximum(_dot(h, w1[...]) + b1[...], 0.0)
    h = jnp.maximum(_dot(h, w2[...]) + b2[...], 0.0)
    o[...] = _dot(h, w3[...]) + b3[...]


def _efin_dec(epre, e, s, ss, dec_ws, dec_bs):
    BE = 8000
    blk = lambda i: (i, 0)
    zero = lambda i: (0, 0)
    w1, w2, w3 = dec_ws
    b1, b2, b3 = dec_bs
    return pl.pallas_call(
        _efin_dec_body,
        grid=(E // BE,),
        in_specs=[pl.BlockSpec((BE, H), blk), pl.BlockSpec((BE, H), blk),
                  pl.BlockSpec((1, H), zero), pl.BlockSpec((1, H), zero),
                  pl.BlockSpec((H, H), zero), pl.BlockSpec((1, H), zero),
                  pl.BlockSpec((H, 64), zero), pl.BlockSpec((1, 64), zero),
                  pl.BlockSpec((64, 3), zero), pl.BlockSpec((1, 3), zero)],
        out_specs=pl.BlockSpec((BE, 3), blk),
        out_shape=jax.ShapeDtypeStruct((E, 3), f32),
    )(epre, e, s, ss, w1, b1.reshape(1, H), w2, b2.reshape(1, 64), w3,
      b3.reshape(1, 3))


def kernel(atom_feats, bond_feats, global_feats, mol_edge_index, rxn_edge_index,
           atom2mol, atom2rxn, params):
    p = params
    srcm = mol_edge_index[0].astype(jnp.int32)
    dstm = mol_edge_index[1].astype(jnp.int32)
    srcr = rxn_edge_index[0].astype(jnp.int32)
    dstr = rxn_edge_index[1].astype(jnp.int32)
    seg_mol = atom2mol.astype(jnp.int32).reshape(N, 1)
    seg_rxn = atom2rxn.astype(jnp.int32).reshape(N, 1)
    zeros_nh = jnp.zeros((N, H), f32)

    gf_pad = jnp.zeros((NSEG, global_feats.shape[1]), f32).at[:M].set(global_feats)

    x, u = _embed_xu(atom_feats, p['emb_atom_W'], p['emb_atom_b'],
                     gf_pad, p['emb_glob_W'], p['emb_glob_b'])
    e = _embed_e(bond_feats, p['emb_bond_W'], p['emb_bond_b'])

    def conv_layer(x, e, u, src, dst, seg2d, cp, need_u):
        gs_t, gd_t, xm_t, ug = _proj4(x, u, cp)
        ug = ug + cp['bx'].reshape(1, H)
        g1, g2, xms = _sc_gather3(gs_t, gd_t, xm_t, src, dst)
        epre, msg, es, ess = _epass(e, g1, g2, xms, cp)
        aggs = _sc_scatter(msg, dst, zeros_nh)
        a0, a1 = aggs[:N], aggs[N:]
        xpre, xs, xss, pool = _xpass_call(x, cp['Wx'], a0, a1, ug, seg2d)
        x_new = _xfin(xpre, x, xs, xss)
        e_new = _efin(epre, e, es, ess)
        u_new = _uupdate(u, pool, cp) if need_u else None
        return x_new, e_new, u_new

    x, e, u = conv_layer(x, e, u, srcm, dstm, seg_mol, p['mol_convs'][0], True)
    x, e, _ = conv_layer(x, e, u, srcm, dstm, seg_mol, p['mol_convs'][1], False)

    u_rxn = _upool(x, seg_rxn)

    x, e, _ = conv_layer(x, e, u_rxn, srcr, dstr, seg_rxn, p['rxn_convs'][0], False)

    cp = p['rxn_convs'][1]
    gs_t, gd_t = _proj2(x, cp)
    g1, g2 = _sc_gather2(gs_t, gd_t, srcr, dstr)
    epre, es, ess = _epass_last(e, g1, g2, cp)
    h = _efin_dec(epre, e, es, ess, p['dec_Ws'], p['dec_bs'])
    return h


# speedup vs baseline: 3.2584x; 1.2510x over previous
import functools

import jax
import jax.numpy as jnp
from jax import lax
from jax.experimental import pallas as pl
from jax.experimental.pallas import tpu as pltpu
from jax.experimental.pallas import tpu_sc as plsc

N = 10000
E = 160000
M = 500
R = 250
H = 128
NSEG = 512
NC, NS = 2, 16
NW = NC * NS
EPW = E // NW
CH = 128
NCH = EPW // CH
TAIL = EPW - NCH * CH
EPS = 1e-5

f32 = jnp.float32


def _mesh():
    return plsc.VectorSubcoreMesh(core_axis_name="c", subcore_axis_name="s",
                                  num_cores=NC, num_subcores=NS)


def _gather_pipelined(tables, idx_sel, n_out):
    ntab = len(tables)

    def body(srch, dsth, outs, isv, idv, bufs, bufsT, semg0, semg1, semw0, semw1):
        wid = lax.axis_index("s") * NC + lax.axis_index("c")
        base = wid * EPW
        pltpu.sync_copy(srch.at[pl.ds(base, EPW)], isv)
        pltpu.sync_copy(dsth.at[pl.ds(base, EPW)], idv)
        ivs = [isv, idv]

        def gstart(i, slot, semg):
            off = i * CH
            for t in range(ntab):
                pltpu.async_copy(
                    tables[t].at[ivs[idx_sel[t]].at[pl.ds(off, CH)]],
                    bufs[t].at[slot], semg)

        def gwait(slot, semg):
            for t in range(ntab):
                pltpu.make_async_copy(
                    tables[t].at[isv.at[pl.ds(0, CH)]], bufs[t].at[slot], semg
                ).wait()

        def wstart(i, slot, semw):
            off = base + i * CH
            for t in range(ntab):
                pltpu.async_copy(bufs[t].at[slot], outs[t].at[pl.ds(off, CH)], semw)

        def wwait(slot, semw):
            for t in range(ntab):
                pltpu.make_async_copy(
                    bufs[t].at[slot], outs[t].at[pl.ds(0, CH)], semw).wait()

        gstart(0, 0, semg0)
        gstart(1, 1, semg1)

        def step(i, carry):
            def for_slot(slot, semg, semw):
                gwait(slot, semg)
                wstart(i, slot, semw)
                wwait(slot, semw)

                @pl.when(i + 2 < NCH)
                def _():
                    gstart(i + 2, slot, semg)

            @pl.when(lax.rem(i, 2) == 0)
            def _():
                for_slot(0, semg0, semw0)

            @pl.when(lax.rem(i, 2) == 1)
            def _():
                for_slot(1, semg1, semw1)

            return carry

        lax.fori_loop(0, NCH, step, 0)

        offT = NCH * CH
        for t in range(ntab):
            pltpu.async_copy(
                tables[t].at[ivs[idx_sel[t]].at[pl.ds(offT, TAIL)]],
                bufsT[t], semg0)
        for t in range(ntab):
            pltpu.make_async_copy(
                tables[t].at[isv.at[pl.ds(0, TAIL)]], bufsT[t], semg0).wait()
        for t in range(ntab):
            pltpu.sync_copy(bufsT[t], outs[t].at[pl.ds(base + offT, TAIL)])

    return body


def _sc_gather_pair(t_pair, t_dst, src, dst):
    @functools.partial(
        pl.kernel,
        out_type=[jax.ShapeDtypeStruct((E, H), jnp.int32),
                  jax.ShapeDtypeStruct((E, H), f32)],
        mesh=_mesh(),
        scratch_types=[
            pltpu.VMEM((EPW,), jnp.int32), pltpu.VMEM((EPW,), jnp.int32),
            pltpu.VMEM((2, CH, H), jnp.int32), pltpu.VMEM((2, CH, H), f32),
            pltpu.VMEM((TAIL, H), jnp.int32), pltpu.VMEM((TAIL, H), f32),
            pltpu.SemaphoreType.DMA, pltpu.SemaphoreType.DMA,
            pltpu.SemaphoreType.DMA, pltpu.SemaphoreType.DMA,
        ],
    )
    def k(t1h, t2h, srch, dsth, gph, g2h,
          isv, idv, b1, b2, b1T, b2T, sg0, sg1, sw0, sw1):
        _gather_pipelined([t1h, t2h], [0, 1], 2)(
            srch, dsth, [gph, g2h], isv, idv,
            [b1, b2], [b1T, b2T], sg0, sg1, sw0, sw1)

    return k(t_pair, t_dst, src, dst)


def _sc_gather2(t_src, t_dst, src, dst):
    @functools.partial(
        pl.kernel,
        out_type=[jax.ShapeDtypeStruct((E, H), f32)] * 2,
        mesh=_mesh(),
        scratch_types=[
            pltpu.VMEM((EPW,), jnp.int32), pltpu.VMEM((EPW,), jnp.int32),
            pltpu.VMEM((2, CH, H), f32), pltpu.VMEM((2, CH, H), f32),
            pltpu.VMEM((TAIL, H), f32), pltpu.VMEM((TAIL, H), f32),
            pltpu.SemaphoreType.DMA, pltpu.SemaphoreType.DMA,
            pltpu.SemaphoreType.DMA, pltpu.SemaphoreType.DMA,
        ],
    )
    def k(t1h, t2h, srch, dsth, g1h, g2h,
          isv, idv, b1, b2, b1T, b2T, sg0, sg1, sw0, sw1):
        _gather_pipelined([t1h, t2h], [0, 1], 2)(
            srch, dsth, [g1h, g2h], isv, idv,
            [b1, b2], [b1T, b2T], sg0, sg1, sw0, sw1)

    return k(t_src, t_dst, src, dst)


def _sc_scatter(msg, dst, zeros_nh):
    @functools.partial(
        pl.kernel,
        out_type=jax.ShapeDtypeStruct((NC * N, H), f32),
        mesh=_mesh(),
        scratch_types=[
            pltpu.VMEM_SHARED((N, H), f32),
            pltpu.VMEM((CH,), jnp.int32), pltpu.VMEM((CH, H), f32),
            pltpu.VMEM((CH,), jnp.int32), pltpu.VMEM((CH, H), f32),
            pltpu.VMEM((TAIL,), jnp.int32), pltpu.VMEM((TAIL, H), f32),
            pltpu.SemaphoreType.DMA, pltpu.SemaphoreType.DMA,
        ],
    )
    def k(msgh, dsth, zh, outh, acc, i0, r0, i1, r1, idxT, rowsT, sm0, sm1):
        cid = lax.axis_index("c")
        sid = lax.axis_index("s")
        wid = sid * NC + cid
        base = wid * EPW

        @pl.when(sid < NS - 1)
        def _():
            pltpu.sync_copy(zh.at[pl.ds(sid * 640, 640)], acc.at[pl.ds(sid * 640, 640)])

        @pl.when(sid == NS - 1)
        def _():
            pltpu.sync_copy(zh.at[pl.ds(9600, 400)], acc.at[pl.ds(9600, 400)])

        plsc.subcore_barrier()

        def lstart(i, iv, rv, sem):
            off = base + i * CH
            pltpu.async_copy(msgh.at[pl.ds(off, CH)], rv, sem)
            pltpu.async_copy(dsth.at[pl.ds(off, CH)], iv, sem)

        def lwait(iv, rv, sem):
            pltpu.make_async_copy(msgh.at[pl.ds(0, CH)], rv, sem).wait()
            pltpu.make_async_copy(dsth.at[pl.ds(0, CH)], iv, sem).wait()

        lstart(0, i0, r0, sm0)

        def step(i, carry):
            def for_slot(iv, rv, sem, iv2, rv2, sem2):
                lwait(iv, rv, sem)

                @pl.when(i + 1 < NCH)
                def _():
                    lstart(i + 1, iv2, rv2, sem2)

                pltpu.sync_copy(rv, acc.at[iv], add=True)

            @pl.when(lax.rem(i, 2) == 0)
            def _():
                for_slot(i0, r0, sm0, i1, r1, sm1)

            @pl.when(lax.rem(i, 2) == 1)
            def _():
                for_slot(i1, r1, sm1, i0, r0, sm0)

            return carry

        lax.fori_loop(0, NCH, step, 0)

        offT = base + NCH * CH
        pltpu.sync_copy(msgh.at[pl.ds(offT, TAIL)], rowsT)
        pltpu.sync_copy(dsth.at[pl.ds(offT, TAIL)], idxT)
        pltpu.sync_copy(rowsT, acc.at[idxT], add=True)

        plsc.subcore_barrier()

        @pl.when(sid < NS - 1)
        def _():
            pltpu.sync_copy(acc.at[pl.ds(sid * 640, 640)],
                            outh.at[pl.ds(cid * N + sid * 640, 640)])

        @pl.when(sid == NS - 1)
        def _():
            pltpu.sync_copy(acc.at[pl.ds(9600, 400)],
                            outh.at[pl.ds(cid * N + 9600, 400)])

    return k(msg, dst, zeros_nh)


def _dot(a, b):
    return jnp.dot(a, b, preferred_element_type=f32)


def _embed_xu_body(af, wa, ba, gf, wg, bg, xo, uo):
    xo[...] = _dot(af[...], wa[...]) + ba[...]
    uo[...] = _dot(gf[...], wg[...]) + bg[...]


def _embed_xu(atom_feats, wa, ba, gf_pad, wg, bg):
    return pl.pallas_call(
        _embed_xu_body,
        out_shape=[jax.ShapeDtypeStruct((N, H), f32),
                   jax.ShapeDtypeStruct((NSEG, H), f32)],
    )(atom_feats, wa, ba.reshape(1, H), gf_pad, wg, bg.reshape(1, H))


def _embed_e_body(bf, w, b, o):
    o[...] = _dot(bf[...], w[...]) + b[...]


def _embed_e(bond_feats, w, b):
    BE = 8000
    return pl.pallas_call(
        _embed_e_body,
        grid=(E // BE,),
        in_specs=[pl.BlockSpec((BE, 64), lambda i: (i, 0)),
                  pl.BlockSpec((64, H), lambda i: (0, 0)),
                  pl.BlockSpec((1, H), lambda i: (0, 0))],
        out_specs=pl.BlockSpec((BE, H), lambda i: (i, 0)),
        out_shape=jax.ShapeDtypeStruct((E, H), f32),
    )(bond_feats, w, b.reshape(1, H))


def _pack_pair(a, b):
    ai = lax.bitcast_convert_type(a, jnp.int32)
    bi = lax.bitcast_convert_type(b, jnp.int32)
    hi = (ai + jnp.int32(0x8000)) & jnp.int32(-65536)
    lo = lax.shift_right_logical(bi + jnp.int32(0x8000), 16)
    return hi | lo


def _unpack_pair(v):
    g1 = lax.bitcast_convert_type(v & jnp.int32(-65536), f32)
    g3 = lax.bitcast_convert_type(lax.shift_left(v, 16), f32)
    return g1, g3


def _proj_pair_body(x, ws, wd, wm, u, wug, bx, gpo, gdo, ugo):
    xv = x[...]
    a = _dot(xv, ws[...])
    b = _dot(xv, wm[...])
    gpo[...] = _pack_pair(a, b)
    gdo[...] = _dot(xv, wd[...])
    ugo[...] = _dot(u[...], wug[...]) + bx[...]


def _proj_pair(x, u, p):
    return pl.pallas_call(
        _proj_pair_body,
        out_shape=[jax.ShapeDtypeStruct((N, H), jnp.int32),
                   jax.ShapeDtypeStruct((N, H), f32),
                   jax.ShapeDtypeStruct((NSEG, H), f32)],
    )(x, p['Wsrc'], p['Wdst'], p['Wm'], u, p['Wug'], p['bx'].reshape(1, H))


def _proj2_body(x, ws, wd, gso, gdo):
    xv = x[...]
    gso[...] = _dot(xv, ws[...])
    gdo[...] = _dot(xv, wd[...])


def _proj2(x, p):
    return pl.pallas_call(
        _proj2_body,
        out_shape=[jax.ShapeDtypeStruct((N, H), f32)] * 2,
    )(x, p['Wsrc'], p['Wdst'])


def _acc_stats(i, epre, s_o, ss_o):
    s = jnp.sum(epre, axis=0, keepdims=True)
    ss = jnp.sum(epre * epre, axis=0, keepdims=True)

    @pl.when(i == 0)
    def _():
        s_o[...] = s
        ss_o[...] = ss

    @pl.when(i != 0)
    def _():
        s_o[...] += s
        ss_o[...] += ss


def _epass1_body(e, gp, g2, we, be, epre_o, msg_o, s_o, ss_o):
    i = pl.program_id(0)
    g1, xms = _unpack_pair(gp[...])
    epre = jnp.maximum(_dot(e[...], we[...]) + g1 + g2[...] + be[...], 0.0)
    epre_o[...] = epre.astype(jnp.bfloat16)
    msg_o[...] = jax.nn.sigmoid(epre) * xms
    _acc_stats(i, epre, s_o, ss_o)


def _epass1(e, gp, g2, p):
    BE = 4000
    blk = lambda i: (i, 0)
    zero = lambda i: (0, 0)
    return pl.pallas_call(
        _epass1_body,
        grid=(E // BE,),
        in_specs=[pl.BlockSpec((BE, H), blk), pl.BlockSpec((BE, H), blk),
                  pl.BlockSpec((BE, H), blk),
                  pl.BlockSpec((H, H), zero), pl.BlockSpec((1, H), zero)],
        out_specs=[pl.BlockSpec((BE, H), blk), pl.BlockSpec((BE, H), blk),
                   pl.BlockSpec((1, H), zero), pl.BlockSpec((1, H), zero)],
        out_shape=[jax.ShapeDtypeStruct((E, H), jnp.bfloat16),
                   jax.ShapeDtypeStruct((E, H), f32),
                   jax.ShapeDtypeStruct((1, H), f32), jax.ShapeDtypeStruct((1, H), f32)],
    )(e, gp, g2, p['We'], p['be'].reshape(1, H))


def _epass_mid_body(epre_p, eold, s_p, ss_p, gp, g2, we, be,
                    enew_o, epre_o, msg_o, s_o, ss_o):
    i = pl.program_id(0)
    mu = s_p[...] / E
    var = ss_p[...] / E - mu * mu
    enew = (epre_p[...].astype(f32) - mu) * lax.rsqrt(var + EPS) + eold[...]
    enew_o[...] = enew
    g1, xms = _unpack_pair(gp[...])
    epre = jnp.maximum(_dot(enew, we[...]) + g1 + g2[...] + be[...], 0.0)
    epre_o[...] = epre.astype(jnp.bfloat16)
    msg_o[...] = jax.nn.sigmoid(epre) * xms
    _acc_stats(i, epre, s_o, ss_o)


def _epass_mid(epre_p, eold, s_p, ss_p, gp, g2, p):
    BE = 4000
    blk = lambda i: (i, 0)
    zero = lambda i: (0, 0)
    return pl.pallas_call(
        _epass_mid_body,
        grid=(E // BE,),
        in_specs=[pl.BlockSpec((BE, H), blk), pl.BlockSpec((BE, H), blk),
                  pl.BlockSpec((1, H), zero), pl.BlockSpec((1, H), zero),
                  pl.BlockSpec((BE, H), blk), pl.BlockSpec((BE, H), blk),
                  pl.BlockSpec((H, H), zero), pl.BlockSpec((1, H), zero)],
        out_specs=[pl.BlockSpec((BE, H), blk), pl.BlockSpec((BE, H), blk),
                   pl.BlockSpec((BE, H), blk),
                   pl.BlockSpec((1, H), zero), pl.BlockSpec((1, H), zero)],
        out_shape=[jax.ShapeDtypeStruct((E, H), f32),
                   jax.ShapeDtypeStruct((E, H), jnp.bfloat16),
                   jax.ShapeDtypeStruct((E, H), f32),
                   jax.ShapeDtypeStruct((1, H), f32), jax.ShapeDtypeStruct((1, H), f32)],
    )(epre_p, eold, s_p, ss_p, gp, g2, p['We'], p['be'].reshape(1, H))


def _epass_last_body(epre_p, eold, s_p, ss_p, g1, g2, we, be,
                     enew_o, epre_o, s_o, ss_o):
    i = pl.program_id(0)
    mu = s_p[...] / E
    var = ss_p[...] / E - mu * mu
    enew = (epre_p[...].astype(f32) - mu) * lax.rsqrt(var + EPS) + eold[...]
    enew_o[...] = enew
    epre = jnp.maximum(_dot(enew, we[...]) + g1[...] + g2[...] + be[...], 0.0)
    epre_o[...] = epre.astype(jnp.bfloat16)
    _acc_stats(i, epre, s_o, ss_o)


def _epass_last(epre_p, eold, s_p, ss_p, g1, g2, p):
    BE = 4000
    blk = lambda i: (i, 0)
    zero = lambda i: (0, 0)
    return pl.pallas_call(
        _epass_last_body,
        grid=(E // BE,),
        in_specs=[pl.BlockSpec((BE, H), blk), pl.BlockSpec((BE, H), blk),
                  pl.BlockSpec((1, H), zero), pl.BlockSpec((1, H), zero),
                  pl.BlockSpec((BE, H), blk), pl.BlockSpec((BE, H), blk),
                  pl.BlockSpec((H, H), zero), pl.BlockSpec((1, H), zero)],
        out_specs=[pl.BlockSpec((BE, H), blk), pl.BlockSpec((BE, H), blk),
                   pl.BlockSpec((1, H), zero), pl.BlockSpec((1, H), zero)],
        out_shape=[jax.ShapeDtypeStruct((E, H), f32),
                   jax.ShapeDtypeStruct((E, H), jnp.bfloat16),
                   jax.ShapeDtypeStruct((1, H), f32), jax.ShapeDtypeStruct((1, H), f32)],
    )(epre_p, eold, s_p, ss_p, g1, g2, p['We'], p['be'].reshape(1, H))


def _xpass_body(bx_, x, wx, a0, a1, ug, seg, xpre_o, s_o, ss_o, pool_o):
    i = pl.program_id(0)
    segv = seg[...]
    oh = (segv == lax.broadcasted_iota(jnp.int32, (bx_, NSEG), 1)).astype(f32)
    ugs = _dot(oh, ug[...])
    xpre = jnp.maximum(_dot(x[...], wx[...]) + a0[...] + a1[...] + ugs, 0.0)
    xpre_o[...] = xpre
    x2 = jnp.concatenate([xpre, jnp.ones((bx_, H), f32)], axis=1)
    pool = lax.dot_general(oh, x2, (((0,), (0,)), ((), ())),
                           preferred_element_type=f32)
    s = jnp.sum(xpre, axis=0, keepdims=True)
    ss = jnp.sum(xpre * xpre, axis=0, keepdims=True)

    @pl.when(i == 0)
    def _():
        s_o[...] = s
        ss_o[...] = ss
        pool_o[...] = pool

    @pl.when(i != 0)
    def _():
        s_o[...] += s
        ss_o[...] += ss
        pool_o[...] += pool


def _xpass_call(x, wx, a0, a1, ug, seg2d):
    B = 2000
    blk = lambda i: (i, 0)
    zero = lambda i: (0, 0)
    return pl.pallas_call(
        functools.partial(_xpass_body, B),
        grid=(N // B,),
        in_specs=[pl.BlockSpec((B, H), blk), pl.BlockSpec((H, H), zero),
                  pl.BlockSpec((B, H), blk), pl.BlockSpec((B, H), blk),
                  pl.BlockSpec((NSEG, H), zero), pl.BlockSpec((B, 1), blk)],
        out_specs=[pl.BlockSpec((B, H), blk),
                   pl.BlockSpec((1, H), zero), pl.BlockSpec((1, H), zero),
                   pl.BlockSpec((NSEG, 2 * H), zero)],
        out_shape=[jax.ShapeDtypeStruct((N, H), f32),
                   jax.ShapeDtypeStruct((1, H), f32), jax.ShapeDtypeStruct((1, H), f32),
                   jax.ShapeDtypeStruct((NSEG, 2 * H), f32)],
    )(x, wx, a0, a1, ug, seg2d)


def _xfin_body(xpre, x, s, ss, o):
    mu = s[...] / N
    var = ss[...] / N - mu * mu
    o[...] = (xpre[...] - mu) * lax.rsqrt(var + EPS) + x[...]


def _xfin(xpre, x, s, ss):
    return pl.pallas_call(
        _xfin_body,
        out_shape=jax.ShapeDtypeStruct((N, H), f32),
    )(xpre, x, s, ss)


def _uupdate_body(u, wu, wg, bu, pool, uo):
    u_pool = pool[:, :H] / jnp.maximum(pool[:, H:], 1.0)
    upre = jnp.maximum(_dot(u[...], wu[...]) + _dot(u_pool, wg[...]) + bu[...], 0.0)
    mask = (lax.broadcasted_iota(jnp.int32, (NSEG, H), 0) < M).astype(f32)
    mu = jnp.sum(upre * mask, axis=0, keepdims=True) / M
    var = jnp.sum(((upre - mu) * mask) ** 2, axis=0, keepdims=True) / M
    uo[...] = (upre - mu) * lax.rsqrt(var + EPS) + u[...]


def _uupdate(u, pool, p):
    return pl.pallas_call(
        _uupdate_body,
        out_shape=jax.ShapeDtypeStruct((NSEG, H), f32),
    )(u, p['Wu'], p['Wg'], p['bu'].reshape(1, H), pool)


def _upool_body(x, seg, uo):
    segv = seg[...]
    oh = (segv == lax.broadcasted_iota(jnp.int32, (N, NSEG), 1)).astype(f32)
    x2 = jnp.concatenate([x[...], jnp.ones((N, H), f32)], axis=1)
    pool = lax.dot_general(oh, x2, (((0,), (0,)), ((), ())),
                           preferred_element_type=f32)
    uo[...] = pool[:, :H] / jnp.maximum(pool[:, H:], 1.0)


def _upool(x, seg2d):
    return pl.pallas_call(
        _upool_body,
        out_shape=jax.ShapeDtypeStruct((NSEG, H), f32),
    )(x, seg2d)


def _efin_dec_body(epre, e, s, ss, w1, b1, w2, b2, w3, b3, o):
    mu = s[...] / E
    var = ss[...] / E - mu * mu
    h = (epre[...].astype(f32) - mu) * lax.rsqrt(var + EPS) + e[...]
    h = jnp.maximum(_dot(h, w1[...]) + b1[...], 0.0)
    h = jnp.maximum(_dot(h, w2[...]) + b2[...], 0.0)
    o[...] = _dot(h, w3[...]) + b3[...]


def _efin_dec(epre, e, s, ss, dec_ws, dec_bs):
    BE = 8000
    blk = lambda i: (i, 0)
    zero = lambda i: (0, 0)
    w1, w2, w3 = dec_ws
    b1, b2, b3 = dec_bs
    return pl.pallas_call(
        _efin_dec_body,
        grid=(E // BE,),
        in_specs=[pl.BlockSpec((BE, H), blk), pl.BlockSpec((BE, H), blk),
                  pl.BlockSpec((1, H), zero), pl.BlockSpec((1, H), zero),
                  pl.BlockSpec((H, H), zero), pl.BlockSpec((1, H), zero),
                  pl.BlockSpec((H, 64), zero), pl.BlockSpec((1, 64), zero),
                  pl.BlockSpec((64, 3), zero), pl.BlockSpec((1, 3), zero)],
        out_specs=pl.BlockSpec((BE, 3), blk),
        out_shape=jax.ShapeDtypeStruct((E, 3), f32),
    )(epre, e, s, ss, w1, b1.reshape(1, H), w2, b2.reshape(1, 64), w3,
      b3.reshape(1, 3))


def kernel(atom_feats, bond_feats, global_feats, mol_edge_index, rxn_edge_index,
           atom2mol, atom2rxn, params):
    p = params
    srcm = mol_edge_index[0].astype(jnp.int32)
    dstm = mol_edge_index[1].astype(jnp.int32)
    srcr = rxn_edge_index[0].astype(jnp.int32)
    dstr = rxn_edge_index[1].astype(jnp.int32)
    seg_mol = atom2mol.astype(jnp.int32).reshape(N, 1)
    seg_rxn = atom2rxn.astype(jnp.int32).reshape(N, 1)
    zeros_nh = jnp.zeros((N, H), f32)

    gf_pad = jnp.zeros((NSEG, global_feats.shape[1]), f32).at[:M].set(global_feats)

    x, u = _embed_xu(atom_feats, p['emb_atom_W'], p['emb_atom_b'],
                     gf_pad, p['emb_glob_W'], p['emb_glob_b'])
    e = _embed_e(bond_feats, p['emb_bond_W'], p['emb_bond_b'])

    def x_side(x, u, msg, ug, dst, seg2d, cp, need_u):
        aggs = _sc_scatter(msg, dst, zeros_nh)
        a0, a1 = aggs[:N], aggs[N:]
        xpre, xs, xss, pool = _xpass_call(x, cp['Wx'], a0, a1, ug, seg2d)
        x_new = _xfin(xpre, x, xs, xss)
        u_new = _uupdate(u, pool, cp) if need_u else None
        return x_new, u_new

    cp = p['mol_convs'][0]
    gp_t, gd_t, ug = _proj_pair(x, u, cp)
    gp, g2 = _sc_gather_pair(gp_t, gd_t, srcm, dstm)
    epre1, msg, s1, ss1 = _epass1(e, gp, g2, cp)
    x, u = x_side(x, u, msg, ug, dstm, seg_mol, cp, True)

    cp = p['mol_convs'][1]
    gp_t, gd_t, ug = _proj_pair(x, u, cp)
    gp, g2 = _sc_gather_pair(gp_t, gd_t, srcm, dstm)
    e1, epre2, msg, s2, ss2 = _epass_mid(epre1, e, s1, ss1, gp, g2, cp)
    x, _ = x_side(x, u, msg, ug, dstm, seg_mol, cp, False)

    u_rxn = _upool(x, seg_rxn)

    cp = p['rxn_convs'][0]
    gp_t, gd_t, ug = _proj_pair(x, u_rxn, cp)
    gp, g2 = _sc_gather_pair(gp_t, gd_t, srcr, dstr)
    e2, epre3, msg, s3, ss3 = _epass_mid(epre2, e1, s2, ss2, gp, g2, cp)
    x, _ = x_side(x, u_rxn, msg, ug, dstr, seg_rxn, cp, False)

    cp = p['rxn_convs'][1]
    gs_t, gd_t = _proj2(x, cp)
    g1, g2 = _sc_gather2(gs_t, gd_t, srcr, dstr)
    e3, epre4, s4, ss4 = _epass_last(epre3, e2, s3, ss3, g1, g2, cp)
    h = _efin_dec(epre4, e3, s4, ss4, p['dec_Ws'], p['dec_bs'])
    return h


# speedup vs baseline: 3.3038x; 1.0139x over previous
import functools

import jax
import jax.numpy as jnp
from jax import lax
from jax.experimental import pallas as pl
from jax.experimental.pallas import tpu as pltpu
from jax.experimental.pallas import tpu_sc as plsc

N = 10000
E = 160000
M = 500
R = 250
H = 128
NSEG = 512
NC, NS = 2, 16
NW = NC * NS
EPW = E // NW
CH = 128
NCH = EPW // CH
TAIL = EPW - NCH * CH
EPS = 1e-5

f32 = jnp.float32


def _mesh():
    return plsc.VectorSubcoreMesh(core_axis_name="c", subcore_axis_name="s",
                                  num_cores=NC, num_subcores=NS)


def _gather_pipelined(tables, idx_sel, n_out):
    ntab = len(tables)

    def body(srch, dsth, outs, isv, idv, bufs, bufsT, semg0, semg1, semw0, semw1):
        wid = lax.axis_index("s") * NC + lax.axis_index("c")
        base = wid * EPW
        pltpu.sync_copy(srch.at[pl.ds(base, EPW)], isv)
        pltpu.sync_copy(dsth.at[pl.ds(base, EPW)], idv)
        ivs = [isv, idv]

        def gstart(i, slot, semg):
            off = i * CH
            for t in range(ntab):
                pltpu.async_copy(
                    tables[t].at[ivs[idx_sel[t]].at[pl.ds(off, CH)]],
                    bufs[t].at[slot], semg)

        def gwait(slot, semg):
            for t in range(ntab):
                pltpu.make_async_copy(
                    tables[t].at[isv.at[pl.ds(0, CH)]], bufs[t].at[slot], semg
                ).wait()

        def wstart(i, slot, semw):
            off = base + i * CH
            for t in range(ntab):
                pltpu.async_copy(bufs[t].at[slot], outs[t].at[pl.ds(off, CH)], semw)

        def wwait(slot, semw):
            for t in range(ntab):
                pltpu.make_async_copy(
                    bufs[t].at[slot], outs[t].at[pl.ds(0, CH)], semw).wait()

        gstart(0, 0, semg0)
        gstart(1, 1, semg1)

        def step(i, carry):
            def for_slot(slot, semg, semw):
                gwait(slot, semg)
                wstart(i, slot, semw)
                wwait(slot, semw)

                @pl.when(i + 2 < NCH)
                def _():
                    gstart(i + 2, slot, semg)

            @pl.when(lax.rem(i, 2) == 0)
            def _():
                for_slot(0, semg0, semw0)

            @pl.when(lax.rem(i, 2) == 1)
            def _():
                for_slot(1, semg1, semw1)

            return carry

        lax.fori_loop(0, NCH, step, 0)

        offT = NCH * CH
        for t in range(ntab):
            pltpu.async_copy(
                tables[t].at[ivs[idx_sel[t]].at[pl.ds(offT, TAIL)]],
                bufsT[t], semg0)
        for t in range(ntab):
            pltpu.make_async_copy(
                tables[t].at[isv.at[pl.ds(0, TAIL)]], bufsT[t], semg0).wait()
        for t in range(ntab):
            pltpu.sync_copy(bufsT[t], outs[t].at[pl.ds(base + offT, TAIL)])

    return body


def _sc_gather_pair(t_pair, t_dst, src, dst):
    @functools.partial(
        pl.kernel,
        out_type=[jax.ShapeDtypeStruct((E, H), jnp.int32),
                  jax.ShapeDtypeStruct((E, H), f32)],
        mesh=_mesh(),
        scratch_types=[
            pltpu.VMEM((EPW,), jnp.int32), pltpu.VMEM((EPW,), jnp.int32),
            pltpu.VMEM((2, CH, H), jnp.int32), pltpu.VMEM((2, CH, H), f32),
            pltpu.VMEM((TAIL, H), jnp.int32), pltpu.VMEM((TAIL, H), f32),
            pltpu.SemaphoreType.DMA, pltpu.SemaphoreType.DMA,
            pltpu.SemaphoreType.DMA, pltpu.SemaphoreType.DMA,
        ],
    )
    def k(t1h, t2h, srch, dsth, gph, g2h,
          isv, idv, b1, b2, b1T, b2T, sg0, sg1, sw0, sw1):
        _gather_pipelined([t1h, t2h], [0, 1], 2)(
            srch, dsth, [gph, g2h], isv, idv,
            [b1, b2], [b1T, b2T], sg0, sg1, sw0, sw1)

    return k(t_pair, t_dst, src, dst)


def _sc_gather2(t_src, t_dst, src, dst):
    @functools.partial(
        pl.kernel,
        out_type=[jax.ShapeDtypeStruct((E, H), f32)] * 2,
        mesh=_mesh(),
        scratch_types=[
            pltpu.VMEM((EPW,), jnp.int32), pltpu.VMEM((EPW,), jnp.int32),
            pltpu.VMEM((2, CH, H), f32), pltpu.VMEM((2, CH, H), f32),
            pltpu.VMEM((TAIL, H), f32), pltpu.VMEM((TAIL, H), f32),
            pltpu.SemaphoreType.DMA, pltpu.SemaphoreType.DMA,
            pltpu.SemaphoreType.DMA, pltpu.SemaphoreType.DMA,
        ],
    )
    def k(t1h, t2h, srch, dsth, g1h, g2h,
          isv, idv, b1, b2, b1T, b2T, sg0, sg1, sw0, sw1):
        _gather_pipelined([t1h, t2h], [0, 1], 2)(
            srch, dsth, [g1h, g2h], isv, idv,
            [b1, b2], [b1T, b2T], sg0, sg1, sw0, sw1)

    return k(t_src, t_dst, src, dst)


def _sc_scatter(msg, dst, zeros_nh):
    @functools.partial(
        pl.kernel,
        out_type=jax.ShapeDtypeStruct((NC * N, H), f32),
        mesh=_mesh(),
        scratch_types=[
            pltpu.VMEM_SHARED((N, H), f32),
            pltpu.VMEM((CH,), jnp.int32), pltpu.VMEM((CH, H), f32),
            pltpu.VMEM((CH,), jnp.int32), pltpu.VMEM((CH, H), f32),
            pltpu.VMEM((TAIL,), jnp.int32), pltpu.VMEM((TAIL, H), f32),
            pltpu.SemaphoreType.DMA, pltpu.SemaphoreType.DMA,
        ],
    )
    def k(msgh, dsth, zh, outh, acc, i0, r0, i1, r1, idxT, rowsT, sm0, sm1):
        cid = lax.axis_index("c")
        sid = lax.axis_index("s")
        wid = sid * NC + cid
        base = wid * EPW

        @pl.when(sid < NS - 1)
        def _():
            pltpu.sync_copy(zh.at[pl.ds(sid * 640, 640)], acc.at[pl.ds(sid * 640, 640)])

        @pl.when(sid == NS - 1)
        def _():
            pltpu.sync_copy(zh.at[pl.ds(9600, 400)], acc.at[pl.ds(9600, 400)])

        plsc.subcore_barrier()

        def lstart(i, iv, rv, sem):
            off = base + i * CH
            pltpu.async_copy(msgh.at[pl.ds(off, CH)], rv, sem)
            pltpu.async_copy(dsth.at[pl.ds(off, CH)], iv, sem)

        def lwait(iv, rv, sem):
            pltpu.make_async_copy(msgh.at[pl.ds(0, CH)], rv, sem).wait()
            pltpu.make_async_copy(dsth.at[pl.ds(0, CH)], iv, sem).wait()

        lstart(0, i0, r0, sm0)

        def step(i, carry):
            def for_slot(iv, rv, sem, iv2, rv2, sem2):
                lwait(iv, rv, sem)

                @pl.when(i + 1 < NCH)
                def _():
                    lstart(i + 1, iv2, rv2, sem2)

                pltpu.sync_copy(rv, acc.at[iv], add=True)

            @pl.when(lax.rem(i, 2) == 0)
            def _():
                for_slot(i0, r0, sm0, i1, r1, sm1)

            @pl.when(lax.rem(i, 2) == 1)
            def _():
                for_slot(i1, r1, sm1, i0, r0, sm0)

            return carry

        lax.fori_loop(0, NCH, step, 0)

        offT = base + NCH * CH
        pltpu.sync_copy(msgh.at[pl.ds(offT, TAIL)], rowsT)
        pltpu.sync_copy(dsth.at[pl.ds(offT, TAIL)], idxT)
        pltpu.sync_copy(rowsT, acc.at[idxT], add=True)

        plsc.subcore_barrier()

        @pl.when(sid < NS - 1)
        def _():
            pltpu.sync_copy(acc.at[pl.ds(sid * 640, 640)],
                            outh.at[pl.ds(cid * N + sid * 640, 640)])

        @pl.when(sid == NS - 1)
        def _():
            pltpu.sync_copy(acc.at[pl.ds(9600, 400)],
                            outh.at[pl.ds(cid * N + 9600, 400)])

    return k(msg, dst, zeros_nh)


def _dot(a, b):
    return jnp.dot(a, b, preferred_element_type=f32)


def _embed_xu_body(af, wa, ba, gf, wg, bg, xo, uo):
    xo[...] = _dot(af[...], wa[...]) + ba[...]
    uo[...] = _dot(gf[...], wg[...]) + bg[...]


def _embed_xu(atom_feats, wa, ba, gf_pad, wg, bg):
    return pl.pallas_call(
        _embed_xu_body,
        out_shape=[jax.ShapeDtypeStruct((N, H), f32),
                   jax.ShapeDtypeStruct((NSEG, H), f32)],
    )(atom_feats, wa, ba.reshape(1, H), gf_pad, wg, bg.reshape(1, H))


def _pack_pair(a, b):
    ai = lax.bitcast_convert_type(a, jnp.int32)
    bi = lax.bitcast_convert_type(b, jnp.int32)
    hi = (ai + jnp.int32(0x8000)) & jnp.int32(-65536)
    lo = lax.shift_right_logical(bi + jnp.int32(0x8000), 16)
    return hi | lo


def _unpack_pair(v):
    g1 = lax.bitcast_convert_type(v & jnp.int32(-65536), f32)
    g3 = lax.bitcast_convert_type(lax.shift_left(v, 16), f32)
    return g1, g3


def _proj_pair_body(x, ws, wd, wm, u, wug, bx, gpo, gdo, ugo):
    xv = x[...]
    a = _dot(xv, ws[...])
    b = _dot(xv, wm[...])
    gpo[...] = _pack_pair(a, b)
    gdo[...] = _dot(xv, wd[...])
    ugo[...] = _dot(u[...], wug[...]) + bx[...]


def _proj_pair(x, u, p):
    return pl.pallas_call(
        _proj_pair_body,
        out_shape=[jax.ShapeDtypeStruct((N, H), jnp.int32),
                   jax.ShapeDtypeStruct((N, H), f32),
                   jax.ShapeDtypeStruct((NSEG, H), f32)],
    )(x, p['Wsrc'], p['Wdst'], p['Wm'], u, p['Wug'], p['bx'].reshape(1, H))


def _proj2_body(x, ws, wd, gso, gdo):
    xv = x[...]
    gso[...] = _dot(xv, ws[...])
    gdo[...] = _dot(xv, wd[...])


def _proj2(x, p):
    return pl.pallas_call(
        _proj2_body,
        out_shape=[jax.ShapeDtypeStruct((N, H), f32)] * 2,
    )(x, p['Wsrc'], p['Wdst'])


def _acc_stats(i, epre, s_o, ss_o):
    s = jnp.sum(epre, axis=0, keepdims=True)
    ss = jnp.sum(epre * epre, axis=0, keepdims=True)

    @pl.when(i == 0)
    def _():
        s_o[...] = s
        ss_o[...] = ss

    @pl.when(i != 0)
    def _():
        s_o[...] += s
        ss_o[...] += ss


def _epass1_body(bf, wemb, bemb, gp, g2, we, be, e0_o, epre_o, msg_o, s_o, ss_o):
    i = pl.program_id(0)
    e0 = _dot(bf[...], wemb[...]) + bemb[...]
    e0_o[...] = e0
    g1, xms = _unpack_pair(gp[...])
    epre = jnp.maximum(_dot(e0, we[...]) + g1 + g2[...] + be[...], 0.0)
    epre_o[...] = epre.astype(jnp.bfloat16)
    msg_o[...] = jax.nn.sigmoid(epre) * xms
    _acc_stats(i, epre, s_o, ss_o)


def _epass1(bond_feats, gp, g2, p, wemb, bemb):
    BE = 4000
    blk = lambda i: (i, 0)
    zero = lambda i: (0, 0)
    return pl.pallas_call(
        _epass1_body,
        grid=(E // BE,),
        in_specs=[pl.BlockSpec((BE, 64), blk), pl.BlockSpec((64, H), zero),
                  pl.BlockSpec((1, H), zero),
                  pl.BlockSpec((BE, H), blk), pl.BlockSpec((BE, H), blk),
                  pl.BlockSpec((H, H), zero), pl.BlockSpec((1, H), zero)],
        out_specs=[pl.BlockSpec((BE, H), blk),
                   pl.BlockSpec((BE, H), blk), pl.BlockSpec((BE, H), blk),
                   pl.BlockSpec((1, H), zero), pl.BlockSpec((1, H), zero)],
        out_shape=[jax.ShapeDtypeStruct((E, H), f32),
                   jax.ShapeDtypeStruct((E, H), jnp.bfloat16),
                   jax.ShapeDtypeStruct((E, H), f32),
                   jax.ShapeDtypeStruct((1, H), f32), jax.ShapeDtypeStruct((1, H), f32)],
    )(bond_feats, wemb, bemb.reshape(1, H), gp, g2, p['We'], p['be'].reshape(1, H))


def _epass_mid_body(epre_p, eold, s_p, ss_p, gp, g2, we, be,
                    enew_o, epre_o, msg_o, s_o, ss_o):
    i = pl.program_id(0)
    mu = s_p[...] / E
    var = ss_p[...] / E - mu * mu
    enew = (epre_p[...].astype(f32) - mu) * lax.rsqrt(var + EPS) + eold[...]
    enew_o[...] = enew
    g1, xms = _unpack_pair(gp[...])
    epre = jnp.maximum(_dot(enew, we[...]) + g1 + g2[...] + be[...], 0.0)
    epre_o[...] = epre.astype(jnp.bfloat16)
    msg_o[...] = jax.nn.sigmoid(epre) * xms
    _acc_stats(i, epre, s_o, ss_o)


def _epass_mid(epre_p, eold, s_p, ss_p, gp, g2, p):
    BE = 4000
    blk = lambda i: (i, 0)
    zero = lambda i: (0, 0)
    return pl.pallas_call(
        _epass_mid_body,
        grid=(E // BE,),
        in_specs=[pl.BlockSpec((BE, H), blk), pl.BlockSpec((BE, H), blk),
                  pl.BlockSpec((1, H), zero), pl.BlockSpec((1, H), zero),
                  pl.BlockSpec((BE, H), blk), pl.BlockSpec((BE, H), blk),
                  pl.BlockSpec((H, H), zero), pl.BlockSpec((1, H), zero)],
        out_specs=[pl.BlockSpec((BE, H), blk), pl.BlockSpec((BE, H), blk),
                   pl.BlockSpec((BE, H), blk),
                   pl.BlockSpec((1, H), zero), pl.BlockSpec((1, H), zero)],
        out_shape=[jax.ShapeDtypeStruct((E, H), f32),
                   jax.ShapeDtypeStruct((E, H), jnp.bfloat16),
                   jax.ShapeDtypeStruct((E, H), f32),
                   jax.ShapeDtypeStruct((1, H), f32), jax.ShapeDtypeStruct((1, H), f32)],
    )(epre_p, eold, s_p, ss_p, gp, g2, p['We'], p['be'].reshape(1, H))


def _epass_last_body(epre_p, eold, s_p, ss_p, g1, g2, we, be,
                     enew_o, epre_o, s_o, ss_o):
    i = pl.program_id(0)
    mu = s_p[...] / E
    var = ss_p[...] / E - mu * mu
    enew = (epre_p[...].astype(f32) - mu) * lax.rsqrt(var + EPS) + eold[...]
    enew_o[...] = enew
    epre = jnp.maximum(_dot(enew, we[...]) + g1[...] + g2[...] + be[...], 0.0)
    epre_o[...] = epre.astype(jnp.bfloat16)
    _acc_stats(i, epre, s_o, ss_o)


def _epass_last(epre_p, eold, s_p, ss_p, g1, g2, p):
    BE = 4000
    blk = lambda i: (i, 0)
    zero = lambda i: (0, 0)
    return pl.pallas_call(
        _epass_last_body,
        grid=(E // BE,),
        in_specs=[pl.BlockSpec((BE, H), blk), pl.BlockSpec((BE, H), blk),
                  pl.BlockSpec((1, H), zero), pl.BlockSpec((1, H), zero),
                  pl.BlockSpec((BE, H), blk), pl.BlockSpec((BE, H), blk),
                  pl.BlockSpec((H, H), zero), pl.BlockSpec((1, H), zero)],
        out_specs=[pl.BlockSpec((BE, H), blk), pl.BlockSpec((BE, H), blk),
                   pl.BlockSpec((1, H), zero), pl.BlockSpec((1, H), zero)],
        out_shape=[jax.ShapeDtypeStruct((E, H), f32),
                   jax.ShapeDtypeStruct((E, H), jnp.bfloat16),
                   jax.ShapeDtypeStruct((1, H), f32), jax.ShapeDtypeStruct((1, H), f32)],
    )(epre_p, eold, s_p, ss_p, g1, g2, p['We'], p['be'].reshape(1, H))


def _xpass_body(bx_, x, wx, a0, a1, ug, seg, xpre_o, s_o, ss_o, pool_o):
    i = pl.program_id(0)
    segv = seg[...]
    oh = (segv == lax.broadcasted_iota(jnp.int32, (bx_, NSEG), 1)).astype(f32)
    ugs = _dot(oh, ug[...])
    xpre = jnp.maximum(_dot(x[...], wx[...]) + a0[...] + a1[...] + ugs, 0.0)
    xpre_o[...] = xpre
    x2 = jnp.concatenate([xpre, jnp.ones((bx_, H), f32)], axis=1)
    pool = lax.dot_general(oh, x2, (((0,), (0,)), ((), ())),
                           preferred_element_type=f32)
    s = jnp.sum(xpre, axis=0, keepdims=True)
    ss = jnp.sum(xpre * xpre, axis=0, keepdims=True)

    @pl.when(i == 0)
    def _():
        s_o[...] = s
        ss_o[...] = ss
        pool_o[...] = pool

    @pl.when(i != 0)
    def _():
        s_o[...] += s
        ss_o[...] += ss
        pool_o[...] += pool


def _xpass_call(x, wx, a0, a1, ug, seg2d):
    B = 2000
    blk = lambda i: (i, 0)
    zero = lambda i: (0, 0)
    return pl.pallas_call(
        functools.partial(_xpass_body, B),
        grid=(N // B,),
        in_specs=[pl.BlockSpec((B, H), blk), pl.BlockSpec((H, H), zero),
                  pl.BlockSpec((B, H), blk), pl.BlockSpec((B, H), blk),
                  pl.BlockSpec((NSEG, H), zero), pl.BlockSpec((B, 1), blk)],
        out_specs=[pl.BlockSpec((B, H), blk),
                   pl.BlockSpec((1, H), zero), pl.BlockSpec((1, H), zero),
                   pl.BlockSpec((NSEG, 2 * H), zero)],
        out_shape=[jax.ShapeDtypeStruct((N, H), f32),
                   jax.ShapeDtypeStruct((1, H), f32), jax.ShapeDtypeStruct((1, H), f32),
                   jax.ShapeDtypeStruct((NSEG, 2 * H), f32)],
    )(x, wx, a0, a1, ug, seg2d)


def _xfin_body(xpre, x, s, ss, o):
    mu = s[...] / N
    var = ss[...] / N - mu * mu
    o[...] = (xpre[...] - mu) * lax.rsqrt(var + EPS) + x[...]


def _xfin(xpre, x, s, ss):
    return pl.pallas_call(
        _xfin_body,
        out_shape=jax.ShapeDtypeStruct((N, H), f32),
    )(xpre, x, s, ss)


def _uupdate_body(u, wu, wg, bu, pool, uo):
    u_pool = pool[:, :H] / jnp.maximum(pool[:, H:], 1.0)
    upre = jnp.maximum(_dot(u[...], wu[...]) + _dot(u_pool, wg[...]) + bu[...], 0.0)
    mask = (lax.broadcasted_iota(jnp.int32, (NSEG, H), 0) < M).astype(f32)
    mu = jnp.sum(upre * mask, axis=0, keepdims=True) / M
    var = jnp.sum(((upre - mu) * mask) ** 2, axis=0, keepdims=True) / M
    uo[...] = (upre - mu) * lax.rsqrt(var + EPS) + u[...]


def _uupdate(u, pool, p):
    return pl.pallas_call(
        _uupdate_body,
        out_shape=jax.ShapeDtypeStruct((NSEG, H), f32),
    )(u, p['Wu'], p['Wg'], p['bu'].reshape(1, H), pool)


def _upool_body(x, seg, uo):
    segv = seg[...]
    oh = (segv == lax.broadcasted_iota(jnp.int32, (N, NSEG), 1)).astype(f32)
    x2 = jnp.concatenate([x[...], jnp.ones((N, H), f32)], axis=1)
    pool = lax.dot_general(oh, x2, (((0,), (0,)), ((), ())),
                           preferred_element_type=f32)
    uo[...] = pool[:, :H] / jnp.maximum(pool[:, H:], 1.0)


def _upool(x, seg2d):
    return pl.pallas_call(
        _upool_body,
        out_shape=jax.ShapeDtypeStruct((NSEG, H), f32),
    )(x, seg2d)


def _efin_dec_body(epre, e, s, ss, w1, b1, w2, b2, w3, b3, o):
    mu = s[...] / E
    var = ss[...] / E - mu * mu
    h = (epre[...].astype(f32) - mu) * lax.rsqrt(var + EPS) + e[...]
    h = jnp.maximum(_dot(h, w1[...]) + b1[...], 0.0)
    h = jnp.maximum(_dot(h, w2[...]) + b2[...], 0.0)
    o[...] = _dot(h, w3[...]) + b3[...]


def _efin_dec(epre, e, s, ss, dec_ws, dec_bs):
    BE = 8000
    blk = lambda i: (i, 0)
    zero = lambda i: (0, 0)
    w1, w2, w3 = dec_ws
    b1, b2, b3 = dec_bs
    return pl.pallas_call(
        _efin_dec_body,
        grid=(E // BE,),
        in_specs=[pl.BlockSpec((BE, H), blk), pl.BlockSpec((BE, H), blk),
                  pl.BlockSpec((1, H), zero), pl.BlockSpec((1, H), zero),
                  pl.BlockSpec((H, H), zero), pl.BlockSpec((1, H), zero),
                  pl.BlockSpec((H, 64), zero), pl.BlockSpec((1, 64), zero),
                  pl.BlockSpec((64, 3), zero), pl.BlockSpec((1, 3), zero)],
        out_specs=pl.BlockSpec((BE, 3), blk),
        out_shape=jax.ShapeDtypeStruct((E, 3), f32),
    )(epre, e, s, ss, w1, b1.reshape(1, H), w2, b2.reshape(1, 64), w3,
      b3.reshape(1, 3))


def kernel(atom_feats, bond_feats, global_feats, mol_edge_index, rxn_edge_index,
           atom2mol, atom2rxn, params):
    p = params
    srcm = mol_edge_index[0].astype(jnp.int32)
    dstm = mol_edge_index[1].astype(jnp.int32)
    srcr = rxn_edge_index[0].astype(jnp.int32)
    dstr = rxn_edge_index[1].astype(jnp.int32)
    seg_mol = atom2mol.astype(jnp.int32).reshape(N, 1)
    seg_rxn = atom2rxn.astype(jnp.int32).reshape(N, 1)
    zeros_nh = jnp.zeros((N, H), f32)

    gf_pad = jnp.zeros((NSEG, global_feats.shape[1]), f32).at[:M].set(global_feats)

    x, u = _embed_xu(atom_feats, p['emb_atom_W'], p['emb_atom_b'],
                     gf_pad, p['emb_glob_W'], p['emb_glob_b'])

    def x_side(x, u, msg, ug, dst, seg2d, cp, need_u):
        aggs = _sc_scatter(msg, dst, zeros_nh)
        a0, a1 = aggs[:N], aggs[N:]
        xpre, xs, xss, pool = _xpass_call(x, cp['Wx'], a0, a1, ug, seg2d)
        x_new = _xfin(xpre, x, xs, xss)
        u_new = _uupdate(u, pool, cp) if need_u else None
        return x_new, u_new

    cp = p['mol_convs'][0]
    gp_t, gd_t, ug = _proj_pair(x, u, cp)
    gp, g2 = _sc_gather_pair(gp_t, gd_t, srcm, dstm)
    e, epre1, msg, s1, ss1 = _epass1(bond_feats, gp, g2, cp,
                                     p['emb_bond_W'], p['emb_bond_b'])
    x, u = x_side(x, u, msg, ug, dstm, seg_mol, cp, True)

    cp = p['mol_convs'][1]
    gp_t, gd_t, ug = _proj_pair(x, u, cp)
    gp, g2 = _sc_gather_pair(gp_t, gd_t, srcm, dstm)
    e1, epre2, msg, s2, ss2 = _epass_mid(epre1, e, s1, ss1, gp, g2, cp)
    x, _ = x_side(x, u, msg, ug, dstm, seg_mol, cp, False)

    u_rxn = _upool(x, seg_rxn)

    cp = p['rxn_convs'][0]
    gp_t, gd_t, ug = _proj_pair(x, u_rxn, cp)
    gp, g2 = _sc_gather_pair(gp_t, gd_t, srcr, dstr)
    e2, epre3, msg, s3, ss3 = _epass_mid(epre2, e1, s2, ss2, gp, g2, cp)
    x, _ = x_side(x, u_rxn, msg, ug, dstr, seg_rxn, cp, False)

    cp = p['rxn_convs'][1]
    gs_t, gd_t = _proj2(x, cp)
    g1, g2 = _sc_gather2(gs_t, gd_t, srcr, dstr)
    e3, epre4, s4, ss4 = _epass_last(epre3, e2, s3, ss3, g1, g2, cp)
    h = _efin_dec(epre4, e3, s4, ss4, p['dec_Ws'], p['dec_bs'])
    return h


# speedup vs baseline: 3.3078x; 1.0012x over previous
import functools

import jax
import jax.numpy as jnp
from jax import lax
from jax.experimental import pallas as pl
from jax.experimental.pallas import tpu as pltpu
from jax.experimental.pallas import tpu_sc as plsc

N = 10000
E = 160000
M = 500
R = 250
H = 128
NSEG = 512
NC, NS = 2, 16
NW = NC * NS
EPW = E // NW
CH = 128
NCH = EPW // CH
TAIL = EPW - NCH * CH
EPS = 1e-5

f32 = jnp.float32


def _mesh():
    return plsc.VectorSubcoreMesh(core_axis_name="c", subcore_axis_name="s",
                                  num_cores=NC, num_subcores=NS)


def _gather_pipelined(tables, idx_sel, n_out):
    ntab = len(tables)

    def body(srch, dsth, outs, isv, idv, bufs, bufsT, semg0, semg1, semw0, semw1):
        wid = lax.axis_index("s") * NC + lax.axis_index("c")
        base = wid * EPW
        pltpu.sync_copy(srch.at[pl.ds(base, EPW)], isv)
        pltpu.sync_copy(dsth.at[pl.ds(base, EPW)], idv)
        ivs = [isv, idv]

        def gstart(i, slot, semg):
            off = i * CH
            for t in range(ntab):
                pltpu.async_copy(
                    tables[t].at[ivs[idx_sel[t]].at[pl.ds(off, CH)]],
                    bufs[t].at[slot], semg)

        def gwait(slot, semg):
            for t in range(ntab):
                pltpu.make_async_copy(
                    tables[t].at[isv.at[pl.ds(0, CH)]], bufs[t].at[slot], semg
                ).wait()

        def wstart(i, slot, semw):
            off = base + i * CH
            for t in range(ntab):
                pltpu.async_copy(bufs[t].at[slot], outs[t].at[pl.ds(off, CH)], semw)

        def wwait(slot, semw):
            for t in range(ntab):
                pltpu.make_async_copy(
                    bufs[t].at[slot], outs[t].at[pl.ds(0, CH)], semw).wait()

        gstart(0, 0, semg0)
        gstart(1, 1, semg1)

        def step(i, carry):
            def for_slot(slot, semg, semw):
                gwait(slot, semg)
                wstart(i, slot, semw)
                wwait(slot, semw)

                @pl.when(i + 2 < NCH)
                def _():
                    gstart(i + 2, slot, semg)

            @pl.when(lax.rem(i, 2) == 0)
            def _():
                for_slot(0, semg0, semw0)

            @pl.when(lax.rem(i, 2) == 1)
            def _():
                for_slot(1, semg1, semw1)

            return carry

        lax.fori_loop(0, NCH, step, 0)

        offT = NCH * CH
        for t in range(ntab):
            pltpu.async_copy(
                tables[t].at[ivs[idx_sel[t]].at[pl.ds(offT, TAIL)]],
                bufsT[t], semg0)
        for t in range(ntab):
            pltpu.make_async_copy(
                tables[t].at[isv.at[pl.ds(0, TAIL)]], bufsT[t], semg0).wait()
        for t in range(ntab):
            pltpu.sync_copy(bufsT[t], outs[t].at[pl.ds(base + offT, TAIL)])

    return body


def _sc_gather_pair(t_pair, t_dst, src, dst):
    @functools.partial(
        pl.kernel,
        out_type=[jax.ShapeDtypeStruct((E, H), jnp.int32),
                  jax.ShapeDtypeStruct((E, H), f32)],
        mesh=_mesh(),
        scratch_types=[
            pltpu.VMEM((EPW,), jnp.int32), pltpu.VMEM((EPW,), jnp.int32),
            pltpu.VMEM((2, CH, H), jnp.int32), pltpu.VMEM((2, CH, H), f32),
            pltpu.VMEM((TAIL, H), jnp.int32), pltpu.VMEM((TAIL, H), f32),
            pltpu.SemaphoreType.DMA, pltpu.SemaphoreType.DMA,
            pltpu.SemaphoreType.DMA, pltpu.SemaphoreType.DMA,
        ],
    )
    def k(t1h, t2h, srch, dsth, gph, g2h,
          isv, idv, b1, b2, b1T, b2T, sg0, sg1, sw0, sw1):
        _gather_pipelined([t1h, t2h], [0, 1], 2)(
            srch, dsth, [gph, g2h], isv, idv,
            [b1, b2], [b1T, b2T], sg0, sg1, sw0, sw1)

    return k(t_pair, t_dst, src, dst)


def _sc_gather2(t_src, t_dst, src, dst):
    @functools.partial(
        pl.kernel,
        out_type=[jax.ShapeDtypeStruct((E, H), f32)] * 2,
        mesh=_mesh(),
        scratch_types=[
            pltpu.VMEM((EPW,), jnp.int32), pltpu.VMEM((EPW,), jnp.int32),
            pltpu.VMEM((2, CH, H), f32), pltpu.VMEM((2, CH, H), f32),
            pltpu.VMEM((TAIL, H), f32), pltpu.VMEM((TAIL, H), f32),
            pltpu.SemaphoreType.DMA, pltpu.SemaphoreType.DMA,
            pltpu.SemaphoreType.DMA, pltpu.SemaphoreType.DMA,
        ],
    )
    def k(t1h, t2h, srch, dsth, g1h, g2h,
          isv, idv, b1, b2, b1T, b2T, sg0, sg1, sw0, sw1):
        _gather_pipelined([t1h, t2h], [0, 1], 2)(
            srch, dsth, [g1h, g2h], isv, idv,
            [b1, b2], [b1T, b2T], sg0, sg1, sw0, sw1)

    return k(t_src, t_dst, src, dst)


def _sc_scatter(msg, dst, zeros_nh):
    @functools.partial(
        pl.kernel,
        out_type=jax.ShapeDtypeStruct((NC * N, H), f32),
        mesh=_mesh(),
        scratch_types=[
            pltpu.VMEM_SHARED((N, H), f32),
            pltpu.VMEM((CH,), jnp.int32), pltpu.VMEM((CH, H), f32),
            pltpu.VMEM((CH,), jnp.int32), pltpu.VMEM((CH, H), f32),
            pltpu.VMEM((TAIL,), jnp.int32), pltpu.VMEM((TAIL, H), f32),
            pltpu.SemaphoreType.DMA, pltpu.SemaphoreType.DMA,
            pltpu.SemaphoreType.DMA, pltpu.SemaphoreType.DMA,
        ],
    )
    def k(msgh, dsth, zh, outh, acc, i0, r0, i1, r1, idxT, rowsT,
          sm0, sm1, sa0, sa1):
        cid = lax.axis_index("c")
        sid = lax.axis_index("s")
        wid = sid * NC + cid
        base = wid * EPW

        @pl.when(sid < NS - 1)
        def _():
            pltpu.sync_copy(zh.at[pl.ds(sid * 640, 640)], acc.at[pl.ds(sid * 640, 640)])

        @pl.when(sid == NS - 1)
        def _():
            pltpu.sync_copy(zh.at[pl.ds(9600, 400)], acc.at[pl.ds(9600, 400)])

        plsc.subcore_barrier()

        def lstart(i, iv, rv, sem):
            off = base + i * CH
            pltpu.async_copy(msgh.at[pl.ds(off, CH)], rv, sem)
            pltpu.async_copy(dsth.at[pl.ds(off, CH)], iv, sem)

        def lwait(iv, rv, sem):
            pltpu.make_async_copy(msgh.at[pl.ds(0, CH)], rv, sem).wait()
            pltpu.make_async_copy(dsth.at[pl.ds(0, CH)], iv, sem).wait()

        lstart(0, i0, r0, sm0)

        def step(i, carry):
            def for_slot(iv, rv, sem, sad, iv2, rv2, sem2, sad2):
                lwait(iv, rv, sem)

                @pl.when(i >= 1)
                def _():
                    pltpu.make_async_copy(rv2, acc.at[iv2], sad2).wait()

                @pl.when(i + 1 < NCH)
                def _():
                    lstart(i + 1, iv2, rv2, sem2)

                pltpu.async_copy(rv, acc.at[iv], sad, add=True)

            @pl.when(lax.rem(i, 2) == 0)
            def _():
                for_slot(i0, r0, sm0, sa0, i1, r1, sm1, sa1)

            @pl.when(lax.rem(i, 2) == 1)
            def _():
                for_slot(i1, r1, sm1, sa1, i0, r0, sm0, sa0)

            return carry

        lax.fori_loop(0, NCH, step, 0)
        pltpu.make_async_copy(r0, acc.at[i0], sa0).wait()

        offT = base + NCH * CH
        pltpu.sync_copy(msgh.at[pl.ds(offT, TAIL)], rowsT)
        pltpu.sync_copy(dsth.at[pl.ds(offT, TAIL)], idxT)
        pltpu.sync_copy(rowsT, acc.at[idxT], add=True)

        plsc.subcore_barrier()

        @pl.when(sid < NS - 1)
        def _():
            pltpu.sync_copy(acc.at[pl.ds(sid * 640, 640)],
                            outh.at[pl.ds(cid * N + sid * 640, 640)])

        @pl.when(sid == NS - 1)
        def _():
            pltpu.sync_copy(acc.at[pl.ds(9600, 400)],
                            outh.at[pl.ds(cid * N + 9600, 400)])

    return k(msg, dst, zeros_nh)


def _dot(a, b):
    return jnp.dot(a, b, preferred_element_type=f32)


def _embed_xu_body(af, wa, ba, gf, wg, bg, xo, uo):
    xo[...] = _dot(af[...], wa[...]) + ba[...]
    uo[...] = _dot(gf[...], wg[...]) + bg[...]


def _embed_xu(atom_feats, wa, ba, gf_pad, wg, bg):
    return pl.pallas_call(
        _embed_xu_body,
        out_shape=[jax.ShapeDtypeStruct((N, H), f32),
                   jax.ShapeDtypeStruct((NSEG, H), f32)],
    )(atom_feats, wa, ba.reshape(1, H), gf_pad, wg, bg.reshape(1, H))


def _pack_pair(a, b):
    ai = lax.bitcast_convert_type(a, jnp.int32)
    bi = lax.bitcast_convert_type(b, jnp.int32)
    hi = (ai + jnp.int32(0x8000)) & jnp.int32(-65536)
    lo = lax.shift_right_logical(bi + jnp.int32(0x8000), 16)
    return hi | lo


def _unpack_pair(v):
    g1 = lax.bitcast_convert_type(v & jnp.int32(-65536), f32)
    g3 = lax.bitcast_convert_type(lax.shift_left(v, 16), f32)
    return g1, g3


def _proj_pair_body(x, ws, wd, wm, u, wug, bx, gpo, gdo, ugo):
    xv = x[...]
    a = _dot(xv, ws[...])
    b = _dot(xv, wm[...])
    gpo[...] = _pack_pair(a, b)
    gdo[...] = _dot(xv, wd[...])
    ugo[...] = _dot(u[...], wug[...]) + bx[...]


def _proj_pair(x, u, p):
    return pl.pallas_call(
        _proj_pair_body,
        out_shape=[jax.ShapeDtypeStruct((N, H), jnp.int32),
                   jax.ShapeDtypeStruct((N, H), f32),
                   jax.ShapeDtypeStruct((NSEG, H), f32)],
    )(x, p['Wsrc'], p['Wdst'], p['Wm'], u, p['Wug'], p['bx'].reshape(1, H))


def _proj2_body(x, ws, wd, gso, gdo):
    xv = x[...]
    gso[...] = _dot(xv, ws[...])
    gdo[...] = _dot(xv, wd[...])


def _proj2(x, p):
    return pl.pallas_call(
        _proj2_body,
        out_shape=[jax.ShapeDtypeStruct((N, H), f32)] * 2,
    )(x, p['Wsrc'], p['Wdst'])


def _acc_stats(i, epre, s_o, ss_o):
    s = jnp.sum(epre, axis=0, keepdims=True)
    ss = jnp.sum(epre * epre, axis=0, keepdims=True)

    @pl.when(i == 0)
    def _():
        s_o[...] = s
        ss_o[...] = ss

    @pl.when(i != 0)
    def _():
        s_o[...] += s
        ss_o[...] += ss


def _epass1_body(bf, wemb, bemb, gp, g2, we, be, e0_o, epre_o, msg_o, s_o, ss_o):
    i = pl.program_id(0)
    e0 = _dot(bf[...], wemb[...]) + bemb[...]
    e0_o[...] = e0
    g1, xms = _unpack_pair(gp[...])
    epre = jnp.maximum(_dot(e0, we[...]) + g1 + g2[...] + be[...], 0.0)
    epre_o[...] = epre.astype(jnp.bfloat16)
    msg_o[...] = jax.nn.sigmoid(epre) * xms
    _acc_stats(i, epre, s_o, ss_o)


def _epass1(bond_feats, gp, g2, p, wemb, bemb):
    BE = 4000
    blk = lambda i: (i, 0)
    zero = lambda i: (0, 0)
    return pl.pallas_call(
        _epass1_body,
        grid=(E // BE,),
        in_specs=[pl.BlockSpec((BE, 64), blk), pl.BlockSpec((64, H), zero),
                  pl.BlockSpec((1, H), zero),
                  pl.BlockSpec((BE, H), blk), pl.BlockSpec((BE, H), blk),
                  pl.BlockSpec((H, H), zero), pl.BlockSpec((1, H), zero)],
        out_specs=[pl.BlockSpec((BE, H), blk),
                   pl.BlockSpec((BE, H), blk), pl.BlockSpec((BE, H), blk),
                   pl.BlockSpec((1, H), zero), pl.BlockSpec((1, H), zero)],
        out_shape=[jax.ShapeDtypeStruct((E, H), f32),
                   jax.ShapeDtypeStruct((E, H), jnp.bfloat16),
                   jax.ShapeDtypeStruct((E, H), f32),
                   jax.ShapeDtypeStruct((1, H), f32), jax.ShapeDtypeStruct((1, H), f32)],
    )(bond_feats, wemb, bemb.reshape(1, H), gp, g2, p['We'], p['be'].reshape(1, H))


def _epass_mid_body(epre_p, eold, s_p, ss_p, gp, g2, we, be,
                    enew_o, epre_o, msg_o, s_o, ss_o):
    i = pl.program_id(0)
    mu = s_p[...] / E
    var = ss_p[...] / E - mu * mu
    enew = (epre_p[...].astype(f32) - mu) * lax.rsqrt(var + EPS) + eold[...]
    enew_o[...] = enew
    g1, xms = _unpack_pair(gp[...])
    epre = jnp.maximum(_dot(enew, we[...]) + g1 + g2[...] + be[...], 0.0)
    epre_o[...] = epre.astype(jnp.bfloat16)
    msg_o[...] = jax.nn.sigmoid(epre) * xms
    _acc_stats(i, epre, s_o, ss_o)


def _epass_mid(epre_p, eold, s_p, ss_p, gp, g2, p):
    BE = 4000
    blk = lambda i: (i, 0)
    zero = lambda i: (0, 0)
    return pl.pallas_call(
        _epass_mid_body,
        grid=(E // BE,),
        in_specs=[pl.BlockSpec((BE, H), blk), pl.BlockSpec((BE, H), blk),
                  pl.BlockSpec((1, H), zero), pl.BlockSpec((1, H), zero),
                  pl.BlockSpec((BE, H), blk), pl.BlockSpec((BE, H), blk),
                  pl.BlockSpec((H, H), zero), pl.BlockSpec((1, H), zero)],
        out_specs=[pl.BlockSpec((BE, H), blk), pl.BlockSpec((BE, H), blk),
                   pl.BlockSpec((BE, H), blk),
                   pl.BlockSpec((1, H), zero), pl.BlockSpec((1, H), zero)],
        out_shape=[jax.ShapeDtypeStruct((E, H), f32),
                   jax.ShapeDtypeStruct((E, H), jnp.bfloat16),
                   jax.ShapeDtypeStruct((E, H), f32),
                   jax.ShapeDtypeStruct((1, H), f32), jax.ShapeDtypeStruct((1, H), f32)],
    )(epre_p, eold, s_p, ss_p, gp, g2, p['We'], p['be'].reshape(1, H))


def _epass_last_body(epre_p, eold, s_p, ss_p, g1, g2, we, be,
                     enew_o, epre_o, s_o, ss_o):
    i = pl.program_id(0)
    mu = s_p[...] / E
    var = ss_p[...] / E - mu * mu
    enew = (epre_p[...].astype(f32) - mu) * lax.rsqrt(var + EPS) + eold[...]
    enew_o[...] = enew
    epre = jnp.maximum(_dot(enew, we[...]) + g1[...] + g2[...] + be[...], 0.0)
    epre_o[...] = epre.astype(jnp.bfloat16)
    _acc_stats(i, epre, s_o, ss_o)


def _epass_last(epre_p, eold, s_p, ss_p, g1, g2, p):
    BE = 4000
    blk = lambda i: (i, 0)
    zero = lambda i: (0, 0)
    return pl.pallas_call(
        _epass_last_body,
        grid=(E // BE,),
        in_specs=[pl.BlockSpec((BE, H), blk), pl.BlockSpec((BE, H), blk),
                  pl.BlockSpec((1, H), zero), pl.BlockSpec((1, H), zero),
                  pl.BlockSpec((BE, H), blk), pl.BlockSpec((BE, H), blk),
                  pl.BlockSpec((H, H), zero), pl.BlockSpec((1, H), zero)],
        out_specs=[pl.BlockSpec((BE, H), blk), pl.BlockSpec((BE, H), blk),
                   pl.BlockSpec((1, H), zero), pl.BlockSpec((1, H), zero)],
        out_shape=[jax.ShapeDtypeStruct((E, H), f32),
                   jax.ShapeDtypeStruct((E, H), jnp.bfloat16),
                   jax.ShapeDtypeStruct((1, H), f32), jax.ShapeDtypeStruct((1, H), f32)],
    )(epre_p, eold, s_p, ss_p, g1, g2, p['We'], p['be'].reshape(1, H))


def _xpass_body(bx_, x, wx, a0, a1, ug, seg, xpre_o, s_o, ss_o, pool_o):
    i = pl.program_id(0)
    segv = seg[...]
    oh = (segv == lax.broadcasted_iota(jnp.int32, (bx_, NSEG), 1)).astype(f32)
    ugs = _dot(oh, ug[...])
    xpre = jnp.maximum(_dot(x[...], wx[...]) + a0[...] + a1[...] + ugs, 0.0)
    xpre_o[...] = xpre
    x2 = jnp.concatenate([xpre, jnp.ones((bx_, H), f32)], axis=1)
    pool = lax.dot_general(oh, x2, (((0,), (0,)), ((), ())),
                           preferred_element_type=f32)
    s = jnp.sum(xpre, axis=0, keepdims=True)
    ss = jnp.sum(xpre * xpre, axis=0, keepdims=True)

    @pl.when(i == 0)
    def _():
        s_o[...] = s
        ss_o[...] = ss
        pool_o[...] = pool

    @pl.when(i != 0)
    def _():
        s_o[...] += s
        ss_o[...] += ss
        pool_o[...] += pool


def _xpass_call(x, wx, a0, a1, ug, seg2d):
    B = 2000
    blk = lambda i: (i, 0)
    zero = lambda i: (0, 0)
    return pl.pallas_call(
        functools.partial(_xpass_body, B),
        grid=(N // B,),
        in_specs=[pl.BlockSpec((B, H), blk), pl.BlockSpec((H, H), zero),
                  pl.BlockSpec((B, H), blk), pl.BlockSpec((B, H), blk),
                  pl.BlockSpec((NSEG, H), zero), pl.BlockSpec((B, 1), blk)],
        out_specs=[pl.BlockSpec((B, H), blk),
                   pl.BlockSpec((1, H), zero), pl.BlockSpec((1, H), zero),
                   pl.BlockSpec((NSEG, 2 * H), zero)],
        out_shape=[jax.ShapeDtypeStruct((N, H), f32),
                   jax.ShapeDtypeStruct((1, H), f32), jax.ShapeDtypeStruct((1, H), f32),
                   jax.ShapeDtypeStruct((NSEG, 2 * H), f32)],
    )(x, wx, a0, a1, ug, seg2d)


def _xfin_body(xpre, x, s, ss, o):
    mu = s[...] / N
    var = ss[...] / N - mu * mu
    o[...] = (xpre[...] - mu) * lax.rsqrt(var + EPS) + x[...]


def _xfin(xpre, x, s, ss):
    return pl.pallas_call(
        _xfin_body,
        out_shape=jax.ShapeDtypeStruct((N, H), f32),
    )(xpre, x, s, ss)


def _uupdate_body(u, wu, wg, bu, pool, uo):
    u_pool = pool[:, :H] / jnp.maximum(pool[:, H:], 1.0)
    upre = jnp.maximum(_dot(u[...], wu[...]) + _dot(u_pool, wg[...]) + bu[...], 0.0)
    mask = (lax.broadcasted_iota(jnp.int32, (NSEG, H), 0) < M).astype(f32)
    mu = jnp.sum(upre * mask, axis=0, keepdims=True) / M
    var = jnp.sum(((upre - mu) * mask) ** 2, axis=0, keepdims=True) / M
    uo[...] = (upre - mu) * lax.rsqrt(var + EPS) + u[...]


def _uupdate(u, pool, p):
    return pl.pallas_call(
        _uupdate_body,
        out_shape=jax.ShapeDtypeStruct((NSEG, H), f32),
    )(u, p['Wu'], p['Wg'], p['bu'].reshape(1, H), pool)


def _upool_body(x, seg, uo):
    segv = seg[...]
    oh = (segv == lax.broadcasted_iota(jnp.int32, (N, NSEG), 1)).astype(f32)
    x2 = jnp.concatenate([x[...], jnp.ones((N, H), f32)], axis=1)
    pool = lax.dot_general(oh, x2, (((0,), (0,)), ((), ())),
                           preferred_element_type=f32)
    uo[...] = pool[:, :H] / jnp.maximum(pool[:, H:], 1.0)


def _upool(x, seg2d):
    return pl.pallas_call(
        _upool_body,
        out_shape=jax.ShapeDtypeStruct((NSEG, H), f32),
    )(x, seg2d)


def _efin_dec_body(epre, e, s, ss, w1, b1, w2, b2, w3, b3, o):
    mu = s[...] / E
    var = ss[...] / E - mu * mu
    h = (epre[...].astype(f32) - mu) * lax.rsqrt(var + EPS) + e[...]
    h = jnp.maximum(_dot(h, w1[...]) + b1[...], 0.0)
    h = jnp.maximum(_dot(h, w2[...]) + b2[...], 0.0)
    o[...] = _dot(h, w3[...]) + b3[...]


def _efin_dec(epre, e, s, ss, dec_ws, dec_bs):
    BE = 8000
    blk = lambda i: (i, 0)
    zero = lambda i: (0, 0)
    w1, w2, w3 = dec_ws
    b1, b2, b3 = dec_bs
    return pl.pallas_call(
        _efin_dec_body,
        grid=(E // BE,),
        in_specs=[pl.BlockSpec((BE, H), blk), pl.BlockSpec((BE, H), blk),
                  pl.BlockSpec((1, H), zero), pl.BlockSpec((1, H), zero),
                  pl.BlockSpec((H, H), zero), pl.BlockSpec((1, H), zero),
                  pl.BlockSpec((H, 64), zero), pl.BlockSpec((1, 64), zero),
                  pl.BlockSpec((64, 3), zero), pl.BlockSpec((1, 3), zero)],
        out_specs=pl.BlockSpec((BE, 3), blk),
        out_shape=jax.ShapeDtypeStruct((E, 3), f32),
    )(epre, e, s, ss, w1, b1.reshape(1, H), w2, b2.reshape(1, 64), w3,
      b3.reshape(1, 3))


def kernel(atom_feats, bond_feats, global_feats, mol_edge_index, rxn_edge_index,
           atom2mol, atom2rxn, params):
    p = params
    srcm = mol_edge_index[0].astype(jnp.int32)
    dstm = mol_edge_index[1].astype(jnp.int32)
    srcr = rxn_edge_index[0].astype(jnp.int32)
    dstr = rxn_edge_index[1].astype(jnp.int32)
    seg_mol = atom2mol.astype(jnp.int32).reshape(N, 1)
    seg_rxn = atom2rxn.astype(jnp.int32).reshape(N, 1)
    zeros_nh = jnp.zeros((N, H), f32)

    gf_pad = jnp.zeros((NSEG, global_feats.shape[1]), f32).at[:M].set(global_feats)

    x, u = _embed_xu(atom_feats, p['emb_atom_W'], p['emb_atom_b'],
                     gf_pad, p['emb_glob_W'], p['emb_glob_b'])

    def x_side(x, u, msg, ug, dst, seg2d, cp, need_u):
        aggs = _sc_scatter(msg, dst, zeros_nh)
        a0, a1 = aggs[:N], aggs[N:]
        xpre, xs, xss, pool = _xpass_call(x, cp['Wx'], a0, a1, ug, seg2d)
        x_new = _xfin(xpre, x, xs, xss)
        u_new = _uupdate(u, pool, cp) if need_u else None
        return x_new, u_new

    cp = p['mol_convs'][0]
    gp_t, gd_t, ug = _proj_pair(x, u, cp)
    gp, g2 = _sc_gather_pair(gp_t, gd_t, srcm, dstm)
    e, epre1, msg, s1, ss1 = _epass1(bond_feats, gp, g2, cp,
                                     p['emb_bond_W'], p['emb_bond_b'])
    x, u = x_side(x, u, msg, ug, dstm, seg_mol, cp, True)

    cp = p['mol_convs'][1]
    gp_t, gd_t, ug = _proj_pair(x, u, cp)
    gp, g2 = _sc_gather_pair(gp_t, gd_t, srcm, dstm)
    e1, epre2, msg, s2, ss2 = _epass_mid(epre1, e, s1, ss1, gp, g2, cp)
    x, _ = x_side(x, u, msg, ug, dstm, seg_mol, cp, False)

    u_rxn = _upool(x, seg_rxn)

    cp = p['rxn_convs'][0]
    gp_t, gd_t, ug = _proj_pair(x, u_rxn, cp)
    gp, g2 = _sc_gather_pair(gp_t, gd_t, srcr, dstr)
    e2, epre3, msg, s3, ss3 = _epass_mid(epre2, e1, s2, ss2, gp, g2, cp)
    x, _ = x_side(x, u_rxn, msg, ug, dstr, seg_rxn, cp, False)

    cp = p['rxn_convs'][1]
    gs_t, gd_t = _proj2(x, cp)
    g1, g2 = _sc_gather2(gs_t, gd_t, srcr, dstr)
    e3, epre4, s4, ss4 = _epass_last(epre3, e2, s3, ss3, g1, g2, cp)
    h = _efin_dec(epre4, e3, s4, ss4, p['dec_Ws'], p['dec_bs'])
    return h


# speedup vs baseline: 3.3372x; 1.0089x over previous
import functools

import jax
import jax.numpy as jnp
from jax import lax
from jax.experimental import pallas as pl
from jax.experimental.pallas import tpu as pltpu
from jax.experimental.pallas import tpu_sc as plsc

N = 10000
E = 160000
M = 500
R = 250
H = 128
NSEG = 512
NC, NS = 2, 16
NW = NC * NS
EPW = E // NW
CH = 128
NCH = EPW // CH
TAIL = EPW - NCH * CH
EPS = 1e-5

f32 = jnp.float32


def _mesh():
    return plsc.VectorSubcoreMesh(core_axis_name="c", subcore_axis_name="s",
                                  num_cores=NC, num_subcores=NS)


def _gather_pipelined(tables, idx_sel, n_out):
    ntab = len(tables)

    def body(srch, dsth, outs, isv, idv, bufs, bufsT, semg0, semg1, semw0, semw1):
        wid = lax.axis_index("s") * NC + lax.axis_index("c")
        base = wid * EPW
        pltpu.sync_copy(srch.at[pl.ds(base, EPW)], isv)
        pltpu.sync_copy(dsth.at[pl.ds(base, EPW)], idv)
        ivs = [isv, idv]

        def gstart(i, slot, semg):
            off = i * CH
            for t in range(ntab):
                pltpu.async_copy(
                    tables[t].at[ivs[idx_sel[t]].at[pl.ds(off, CH)]],
                    bufs[t].at[slot], semg)

        def gwait(slot, semg):
            for t in range(ntab):
                pltpu.make_async_copy(
                    tables[t].at[isv.at[pl.ds(0, CH)]], bufs[t].at[slot], semg
                ).wait()

        def wstart(i, slot, semw):
            off = base + i * CH
            for t in range(ntab):
                pltpu.async_copy(bufs[t].at[slot], outs[t].at[pl.ds(off, CH)], semw)

        def wwait(slot, semw):
            for t in range(ntab):
                pltpu.make_async_copy(
                    bufs[t].at[slot], outs[t].at[pl.ds(0, CH)], semw).wait()

        gstart(0, 0, semg0)
        gstart(1, 1, semg1)

        def step(i, carry):
            def for_slot(slot, semg, semw):
                gwait(slot, semg)
                wstart(i, slot, semw)
                wwait(slot, semw)

                @pl.when(i + 2 < NCH)
                def _():
                    gstart(i + 2, slot, semg)

            @pl.when(lax.rem(i, 2) == 0)
            def _():
                for_slot(0, semg0, semw0)

            @pl.when(lax.rem(i, 2) == 1)
            def _():
                for_slot(1, semg1, semw1)

            return carry

        lax.fori_loop(0, NCH, step, 0)

        offT = NCH * CH
        for t in range(ntab):
            pltpu.async_copy(
                tables[t].at[ivs[idx_sel[t]].at[pl.ds(offT, TAIL)]],
                bufsT[t], semg0)
        for t in range(ntab):
            pltpu.make_async_copy(
                tables[t].at[isv.at[pl.ds(0, TAIL)]], bufsT[t], semg0).wait()
        for t in range(ntab):
            pltpu.sync_copy(bufsT[t], outs[t].at[pl.ds(base + offT, TAIL)])

    return body


def _sc_gather_pair(t_pair, t_dst, src, dst):
    @functools.partial(
        pl.kernel,
        out_type=[jax.ShapeDtypeStruct((E, H), jnp.int32),
                  jax.ShapeDtypeStruct((E, H), f32)],
        mesh=_mesh(),
        scratch_types=[
            pltpu.VMEM((EPW,), jnp.int32), pltpu.VMEM((EPW,), jnp.int32),
            pltpu.VMEM((2, CH, H), jnp.int32), pltpu.VMEM((2, CH, H), f32),
            pltpu.VMEM((TAIL, H), jnp.int32), pltpu.VMEM((TAIL, H), f32),
            pltpu.SemaphoreType.DMA, pltpu.SemaphoreType.DMA,
            pltpu.SemaphoreType.DMA, pltpu.SemaphoreType.DMA,
        ],
    )
    def k(t1h, t2h, srch, dsth, gph, g2h,
          isv, idv, b1, b2, b1T, b2T, sg0, sg1, sw0, sw1):
        _gather_pipelined([t1h, t2h], [0, 1], 2)(
            srch, dsth, [gph, g2h], isv, idv,
            [b1, b2], [b1T, b2T], sg0, sg1, sw0, sw1)

    return k(t_pair, t_dst, src, dst)


def _sc_gather2(t_src, t_dst, src, dst):
    @functools.partial(
        pl.kernel,
        out_type=[jax.ShapeDtypeStruct((E, H), f32)] * 2,
        mesh=_mesh(),
        scratch_types=[
            pltpu.VMEM((EPW,), jnp.int32), pltpu.VMEM((EPW,), jnp.int32),
            pltpu.VMEM((2, CH, H), f32), pltpu.VMEM((2, CH, H), f32),
            pltpu.VMEM((TAIL, H), f32), pltpu.VMEM((TAIL, H), f32),
            pltpu.SemaphoreType.DMA, pltpu.SemaphoreType.DMA,
            pltpu.SemaphoreType.DMA, pltpu.SemaphoreType.DMA,
        ],
    )
    def k(t1h, t2h, srch, dsth, g1h, g2h,
          isv, idv, b1, b2, b1T, b2T, sg0, sg1, sw0, sw1):
        _gather_pipelined([t1h, t2h], [0, 1], 2)(
            srch, dsth, [g1h, g2h], isv, idv,
            [b1, b2], [b1T, b2T], sg0, sg1, sw0, sw1)

    return k(t_src, t_dst, src, dst)


def _sc_scatter(msg, dst, zeros_nh):
    @functools.partial(
        pl.kernel,
        out_type=jax.ShapeDtypeStruct((NC * N, H), f32),
        mesh=_mesh(),
        scratch_types=[
            pltpu.VMEM_SHARED((N, H), f32),
            pltpu.VMEM((CH,), jnp.int32), pltpu.VMEM((CH, H), f32),
            pltpu.VMEM((CH,), jnp.int32), pltpu.VMEM((CH, H), f32),
            pltpu.VMEM((TAIL,), jnp.int32), pltpu.VMEM((TAIL, H), f32),
            pltpu.SemaphoreType.DMA, pltpu.SemaphoreType.DMA,
            pltpu.SemaphoreType.DMA, pltpu.SemaphoreType.DMA,
        ],
    )
    def k(msgh, dsth, zh, outh, acc, i0, r0, i1, r1, idxT, rowsT,
          sm0, sm1, sa0, sa1):
        cid = lax.axis_index("c")
        sid = lax.axis_index("s")
        wid = sid * NC + cid
        base = wid * EPW

        @pl.when(sid < NS - 1)
        def _():
            pltpu.sync_copy(zh.at[pl.ds(sid * 640, 640)], acc.at[pl.ds(sid * 640, 640)])

        @pl.when(sid == NS - 1)
        def _():
            pltpu.sync_copy(zh.at[pl.ds(9600, 400)], acc.at[pl.ds(9600, 400)])

        plsc.subcore_barrier()

        def lstart(i, iv, rv, sem):
            off = base + i * CH
            pltpu.async_copy(msgh.at[pl.ds(off, CH)], rv, sem)
            pltpu.async_copy(dsth.at[pl.ds(off, CH)], iv, sem)

        def lwait(iv, rv, sem):
            pltpu.make_async_copy(msgh.at[pl.ds(0, CH)], rv, sem).wait()
            pltpu.make_async_copy(dsth.at[pl.ds(0, CH)], iv, sem).wait()

        lstart(0, i0, r0, sm0)

        def step(i, carry):
            def for_slot(iv, rv, sem, sad, iv2, rv2, sem2, sad2):
                lwait(iv, rv, sem)

                @pl.when(i >= 1)
                def _():
                    pltpu.make_async_copy(rv2, acc.at[iv2], sad2).wait()

                @pl.when(i + 1 < NCH)
                def _():
                    lstart(i + 1, iv2, rv2, sem2)

                pltpu.async_copy(rv, acc.at[iv], sad, add=True)

            @pl.when(lax.rem(i, 2) == 0)
            def _():
                for_slot(i0, r0, sm0, sa0, i1, r1, sm1, sa1)

            @pl.when(lax.rem(i, 2) == 1)
            def _():
                for_slot(i1, r1, sm1, sa1, i0, r0, sm0, sa0)

            return carry

        lax.fori_loop(0, NCH, step, 0)
        pltpu.make_async_copy(r0, acc.at[i0], sa0).wait()

        offT = base + NCH * CH
        pltpu.sync_copy(msgh.at[pl.ds(offT, TAIL)], rowsT)
        pltpu.sync_copy(dsth.at[pl.ds(offT, TAIL)], idxT)
        pltpu.sync_copy(rowsT, acc.at[idxT], add=True)

        plsc.subcore_barrier()

        @pl.when(sid < NS - 1)
        def _():
            pltpu.sync_copy(acc.at[pl.ds(sid * 640, 640)],
                            outh.at[pl.ds(cid * N + sid * 640, 640)])

        @pl.when(sid == NS - 1)
        def _():
            pltpu.sync_copy(acc.at[pl.ds(9600, 400)],
                            outh.at[pl.ds(cid * N + 9600, 400)])

    return k(msg, dst, zeros_nh)


def _dot(a, b):
    return jnp.dot(a, b, preferred_element_type=f32)


def _embed_xu_body(af, wa, ba, gf, wg, bg, xo, uo):
    xo[...] = _dot(af[...], wa[...]) + ba[...]
    uo[...] = _dot(gf[...], wg[...]) + bg[...]


def _embed_xu(atom_feats, wa, ba, gf_pad, wg, bg):
    return pl.pallas_call(
        _embed_xu_body,
        out_shape=[jax.ShapeDtypeStruct((N, H), f32),
                   jax.ShapeDtypeStruct((NSEG, H), f32)],
    )(atom_feats, wa, ba.reshape(1, H), gf_pad, wg, bg.reshape(1, H))


def _pack_pair(a, b):
    ai = lax.bitcast_convert_type(a, jnp.int32)
    bi = lax.bitcast_convert_type(b, jnp.int32)
    hi = (ai + jnp.int32(0x8000)) & jnp.int32(-65536)
    lo = lax.shift_right_logical(bi + jnp.int32(0x8000), 16)
    return hi | lo


def _unpack_pair(v):
    g1 = lax.bitcast_convert_type(v & jnp.int32(-65536), f32)
    g3 = lax.bitcast_convert_type(lax.shift_left(v, 16), f32)
    return g1, g3


def _proj_pair_body(x, ws, wd, wm, u, wug, bx, gpo, gdo, ugo):
    xv = x[...]
    a = _dot(xv, ws[...])
    b = _dot(xv, wm[...])
    gpo[...] = _pack_pair(a, b)
    gdo[...] = _dot(xv, wd[...])
    ugo[...] = _dot(u[...], wug[...]) + bx[...]


def _proj_pair(x, u, p):
    return pl.pallas_call(
        _proj_pair_body,
        out_shape=[jax.ShapeDtypeStruct((N, H), jnp.int32),
                   jax.ShapeDtypeStruct((N, H), f32),
                   jax.ShapeDtypeStruct((NSEG, H), f32)],
    )(x, p['Wsrc'], p['Wdst'], p['Wm'], u, p['Wug'], p['bx'].reshape(1, H))


def _proj2_body(x, ws, wd, gso, gdo):
    xv = x[...]
    gso[...] = _dot(xv, ws[...])
    gdo[...] = _dot(xv, wd[...])


def _proj2(x, p):
    return pl.pallas_call(
        _proj2_body,
        out_shape=[jax.ShapeDtypeStruct((N, H), f32)] * 2,
    )(x, p['Wsrc'], p['Wdst'])


def _acc_stats(i, epre, s_o, ss_o):
    s = jnp.sum(epre, axis=0, keepdims=True)
    ss = jnp.sum(epre * epre, axis=0, keepdims=True)

    @pl.when(i == 0)
    def _():
        s_o[...] = s
        ss_o[...] = ss

    @pl.when(i != 0)
    def _():
        s_o[...] += s
        ss_o[...] += ss


def _epass1_body(bf, wemb, bemb, gp, g2, we, be, epre_o, msg_o, s_o, ss_o):
    i = pl.program_id(0)
    e0 = _dot(bf[...], wemb[...]) + bemb[...]
    g1, xms = _unpack_pair(gp[...])
    epre = jnp.maximum(_dot(e0, we[...]) + g1 + g2[...] + be[...], 0.0)
    epre_o[...] = epre.astype(jnp.bfloat16)
    msg_o[...] = jax.nn.sigmoid(epre) * xms
    _acc_stats(i, epre, s_o, ss_o)


def _epass1(bond_feats, gp, g2, p, wemb, bemb):
    BE = 4000
    blk = lambda i: (i, 0)
    zero = lambda i: (0, 0)
    return pl.pallas_call(
        _epass1_body,
        grid=(E // BE,),
        in_specs=[pl.BlockSpec((BE, 64), blk), pl.BlockSpec((64, H), zero),
                  pl.BlockSpec((1, H), zero),
                  pl.BlockSpec((BE, H), blk), pl.BlockSpec((BE, H), blk),
                  pl.BlockSpec((H, H), zero), pl.BlockSpec((1, H), zero)],
        out_specs=[pl.BlockSpec((BE, H), blk), pl.BlockSpec((BE, H), blk),
                   pl.BlockSpec((1, H), zero), pl.BlockSpec((1, H), zero)],
        out_shape=[jax.ShapeDtypeStruct((E, H), jnp.bfloat16),
                   jax.ShapeDtypeStruct((E, H), f32),
                   jax.ShapeDtypeStruct((1, H), f32), jax.ShapeDtypeStruct((1, H), f32)],
    )(bond_feats, wemb, bemb.reshape(1, H), gp, g2, p['We'], p['be'].reshape(1, H))


def _epass_mid2_body(epre_p, bf, wemb, bemb, s_p, ss_p, gp, g2, we, be,
                     enew_o, epre_o, msg_o, s_o, ss_o):
    i = pl.program_id(0)
    mu = s_p[...] / E
    var = ss_p[...] / E - mu * mu
    e0 = _dot(bf[...], wemb[...]) + bemb[...]
    enew = (epre_p[...].astype(f32) - mu) * lax.rsqrt(var + EPS) + e0
    enew_o[...] = enew
    g1, xms = _unpack_pair(gp[...])
    epre = jnp.maximum(_dot(enew, we[...]) + g1 + g2[...] + be[...], 0.0)
    epre_o[...] = epre.astype(jnp.bfloat16)
    msg_o[...] = jax.nn.sigmoid(epre) * xms
    _acc_stats(i, epre, s_o, ss_o)


def _epass_mid2(epre_p, bond_feats, wemb, bemb, s_p, ss_p, gp, g2, p):
    BE = 4000
    blk = lambda i: (i, 0)
    zero = lambda i: (0, 0)
    return pl.pallas_call(
        _epass_mid2_body,
        grid=(E // BE,),
        in_specs=[pl.BlockSpec((BE, H), blk), pl.BlockSpec((BE, 64), blk),
                  pl.BlockSpec((64, H), zero), pl.BlockSpec((1, H), zero),
                  pl.BlockSpec((1, H), zero), pl.BlockSpec((1, H), zero),
                  pl.BlockSpec((BE, H), blk), pl.BlockSpec((BE, H), blk),
                  pl.BlockSpec((H, H), zero), pl.BlockSpec((1, H), zero)],
        out_specs=[pl.BlockSpec((BE, H), blk), pl.BlockSpec((BE, H), blk),
                   pl.BlockSpec((BE, H), blk),
                   pl.BlockSpec((1, H), zero), pl.BlockSpec((1, H), zero)],
        out_shape=[jax.ShapeDtypeStruct((E, H), f32),
                   jax.ShapeDtypeStruct((E, H), jnp.bfloat16),
                   jax.ShapeDtypeStruct((E, H), f32),
                   jax.ShapeDtypeStruct((1, H), f32), jax.ShapeDtypeStruct((1, H), f32)],
    )(epre_p, bond_feats, wemb, bemb.reshape(1, H), s_p, ss_p, gp, g2,
      p['We'], p['be'].reshape(1, H))


def _epass_mid_body(epre_p, eold, s_p, ss_p, gp, g2, we, be,
                    enew_o, epre_o, msg_o, s_o, ss_o):
    i = pl.program_id(0)
    mu = s_p[...] / E
    var = ss_p[...] / E - mu * mu
    enew = (epre_p[...].astype(f32) - mu) * lax.rsqrt(var + EPS) + eold[...]
    enew_o[...] = enew
    g1, xms = _unpack_pair(gp[...])
    epre = jnp.maximum(_dot(enew, we[...]) + g1 + g2[...] + be[...], 0.0)
    epre_o[...] = epre.astype(jnp.bfloat16)
    msg_o[...] = jax.nn.sigmoid(epre) * xms
    _acc_stats(i, epre, s_o, ss_o)


def _epass_mid(epre_p, eold, s_p, ss_p, gp, g2, p):
    BE = 4000
    blk = lambda i: (i, 0)
    zero = lambda i: (0, 0)
    return pl.pallas_call(
        _epass_mid_body,
        grid=(E // BE,),
        in_specs=[pl.BlockSpec((BE, H), blk), pl.BlockSpec((BE, H), blk),
                  pl.BlockSpec((1, H), zero), pl.BlockSpec((1, H), zero),
                  pl.BlockSpec((BE, H), blk), pl.BlockSpec((BE, H), blk),
                  pl.BlockSpec((H, H), zero), pl.BlockSpec((1, H), zero)],
        out_specs=[pl.BlockSpec((BE, H), blk), pl.BlockSpec((BE, H), blk),
                   pl.BlockSpec((BE, H), blk),
                   pl.BlockSpec((1, H), zero), pl.BlockSpec((1, H), zero)],
        out_shape=[jax.ShapeDtypeStruct((E, H), f32),
                   jax.ShapeDtypeStruct((E, H), jnp.bfloat16),
                   jax.ShapeDtypeStruct((E, H), f32),
                   jax.ShapeDtypeStruct((1, H), f32), jax.ShapeDtypeStruct((1, H), f32)],
    )(epre_p, eold, s_p, ss_p, gp, g2, p['We'], p['be'].reshape(1, H))


def _epass_last_body(epre_p, eold, s_p, ss_p, g1, g2, we, be,
                     enew_o, epre_o, s_o, ss_o):
    i = pl.program_id(0)
    mu = s_p[...] / E
    var = ss_p[...] / E - mu * mu
    enew = (epre_p[...].astype(f32) - mu) * lax.rsqrt(var + EPS) + eold[...]
    enew_o[...] = enew
    epre = jnp.maximum(_dot(enew, we[...]) + g1[...] + g2[...] + be[...], 0.0)
    epre_o[...] = epre.astype(jnp.bfloat16)
    _acc_stats(i, epre, s_o, ss_o)


def _epass_last(epre_p, eold, s_p, ss_p, g1, g2, p):
    BE = 4000
    blk = lambda i: (i, 0)
    zero = lambda i: (0, 0)
    return pl.pallas_call(
        _epass_last_body,
        grid=(E // BE,),
        in_specs=[pl.BlockSpec((BE, H), blk), pl.BlockSpec((BE, H), blk),
                  pl.BlockSpec((1, H), zero), pl.BlockSpec((1, H), zero),
                  pl.BlockSpec((BE, H), blk), pl.BlockSpec((BE, H), blk),
                  pl.BlockSpec((H, H), zero), pl.BlockSpec((1, H), zero)],
        out_specs=[pl.BlockSpec((BE, H), blk), pl.BlockSpec((BE, H), blk),
                   pl.BlockSpec((1, H), zero), pl.BlockSpec((1, H), zero)],
        out_shape=[jax.ShapeDtypeStruct((E, H), f32),
                   jax.ShapeDtypeStruct((E, H), jnp.bfloat16),
                   jax.ShapeDtypeStruct((1, H), f32), jax.ShapeDtypeStruct((1, H), f32)],
    )(epre_p, eold, s_p, ss_p, g1, g2, p['We'], p['be'].reshape(1, H))


def _xpass_body(bx_, x, wx, a0, a1, ug, seg, xpre_o, s_o, ss_o, pool_o):
    i = pl.program_id(0)
    segv = seg[...]
    oh = (segv == lax.broadcasted_iota(jnp.int32, (bx_, NSEG), 1)).astype(f32)
    ugs = _dot(oh, ug[...])
    xpre = jnp.maximum(_dot(x[...], wx[...]) + a0[...] + a1[...] + ugs, 0.0)
    xpre_o[...] = xpre
    x2 = jnp.concatenate([xpre, jnp.ones((bx_, H), f32)], axis=1)
    pool = lax.dot_general(oh, x2, (((0,), (0,)), ((), ())),
                           preferred_element_type=f32)
    s = jnp.sum(xpre, axis=0, keepdims=True)
    ss = jnp.sum(xpre * xpre, axis=0, keepdims=True)

    @pl.when(i == 0)
    def _():
        s_o[...] = s
        ss_o[...] = ss
        pool_o[...] = pool

    @pl.when(i != 0)
    def _():
        s_o[...] += s
        ss_o[...] += ss
        pool_o[...] += pool


def _xpass_call(x, wx, a0, a1, ug, seg2d):
    B = 2000
    blk = lambda i: (i, 0)
    zero = lambda i: (0, 0)
    return pl.pallas_call(
        functools.partial(_xpass_body, B),
        grid=(N // B,),
        in_specs=[pl.BlockSpec((B, H), blk), pl.BlockSpec((H, H), zero),
                  pl.BlockSpec((B, H), blk), pl.BlockSpec((B, H), blk),
                  pl.BlockSpec((NSEG, H), zero), pl.BlockSpec((B, 1), blk)],
        out_specs=[pl.BlockSpec((B, H), blk),
                   pl.BlockSpec((1, H), zero), pl.BlockSpec((1, H), zero),
                   pl.BlockSpec((NSEG, 2 * H), zero)],
        out_shape=[jax.ShapeDtypeStruct((N, H), f32),
                   jax.ShapeDtypeStruct((1, H), f32), jax.ShapeDtypeStruct((1, H), f32),
                   jax.ShapeDtypeStruct((NSEG, 2 * H), f32)],
    )(x, wx, a0, a1, ug, seg2d)


def _xfin_body(xpre, x, s, ss, o):
    mu = s[...] / N
    var = ss[...] / N - mu * mu
    o[...] = (xpre[...] - mu) * lax.rsqrt(var + EPS) + x[...]


def _xfin(xpre, x, s, ss):
    return pl.pallas_call(
        _xfin_body,
        out_shape=jax.ShapeDtypeStruct((N, H), f32),
    )(xpre, x, s, ss)


def _uupdate_body(u, wu, wg, bu, pool, uo):
    u_pool = pool[:, :H] / jnp.maximum(pool[:, H:], 1.0)
    upre = jnp.maximum(_dot(u[...], wu[...]) + _dot(u_pool, wg[...]) + bu[...], 0.0)
    mask = (lax.broadcasted_iota(jnp.int32, (NSEG, H), 0) < M).astype(f32)
    mu = jnp.sum(upre * mask, axis=0, keepdims=True) / M
    var = jnp.sum(((upre - mu) * mask) ** 2, axis=0, keepdims=True) / M
    uo[...] = (upre - mu) * lax.rsqrt(var + EPS) + u[...]


def _uupdate(u, pool, p):
    return pl.pallas_call(
        _uupdate_body,
        out_shape=jax.ShapeDtypeStruct((NSEG, H), f32),
    )(u, p['Wu'], p['Wg'], p['bu'].reshape(1, H), pool)


def _upool_body(x, seg, uo):
    segv = seg[...]
    oh = (segv == lax.broadcasted_iota(jnp.int32, (N, NSEG), 1)).astype(f32)
    x2 = jnp.concatenate([x[...], jnp.ones((N, H), f32)], axis=1)
    pool = lax.dot_general(oh, x2, (((0,), (0,)), ((), ())),
                           preferred_element_type=f32)
    uo[...] = pool[:, :H] / jnp.maximum(pool[:, H:], 1.0)


def _upool(x, seg2d):
    return pl.pallas_call(
        _upool_body,
        out_shape=jax.ShapeDtypeStruct((NSEG, H), f32),
    )(x, seg2d)


def _efin_dec_body(epre, e, s, ss, w1, b1, w2, b2, w3, b3, o):
    mu = s[...] / E
    var = ss[...] / E - mu * mu
    h = (epre[...].astype(f32) - mu) * lax.rsqrt(var + EPS) + e[...]
    h = jnp.maximum(_dot(h, w1[...]) + b1[...], 0.0)
    h = jnp.maximum(_dot(h, w2[...]) + b2[...], 0.0)
    o[...] = _dot(h, w3[...]) + b3[...]


def _efin_dec(epre, e, s, ss, dec_ws, dec_bs):
    BE = 8000
    blk = lambda i: (i, 0)
    zero = lambda i: (0, 0)
    w1, w2, w3 = dec_ws
    b1, b2, b3 = dec_bs
    return pl.pallas_call(
        _efin_dec_body,
        grid=(E // BE,),
        in_specs=[pl.BlockSpec((BE, H), blk), pl.BlockSpec((BE, H), blk),
                  pl.BlockSpec((1, H), zero), pl.BlockSpec((1, H), zero),
                  pl.BlockSpec((H, H), zero), pl.BlockSpec((1, H), zero),
                  pl.BlockSpec((H, 64), zero), pl.BlockSpec((1, 64), zero),
                  pl.BlockSpec((64, 3), zero), pl.BlockSpec((1, 3), zero)],
        out_specs=pl.BlockSpec((BE, 3), blk),
        out_shape=jax.ShapeDtypeStruct((E, 3), f32),
    )(epre, e, s, ss, w1, b1.reshape(1, H), w2, b2.reshape(1, 64), w3,
      b3.reshape(1, 3))


def kernel(atom_feats, bond_feats, global_feats, mol_edge_index, rxn_edge_index,
           atom2mol, atom2rxn, params):
    p = params
    srcm = mol_edge_index[0].astype(jnp.int32)
    dstm = mol_edge_index[1].astype(jnp.int32)
    srcr = rxn_edge_index[0].astype(jnp.int32)
    dstr = rxn_edge_index[1].astype(jnp.int32)
    seg_mol = atom2mol.astype(jnp.int32).reshape(N, 1)
    seg_rxn = atom2rxn.astype(jnp.int32).reshape(N, 1)
    zeros_nh = jnp.zeros((N, H), f32)

    gf_pad = jnp.zeros((NSEG, global_feats.shape[1]), f32).at[:M].set(global_feats)

    x, u = _embed_xu(atom_feats, p['emb_atom_W'], p['emb_atom_b'],
                     gf_pad, p['emb_glob_W'], p['emb_glob_b'])

    def x_side(x, u, msg, ug, dst, seg2d, cp, need_u):
        aggs = _sc_scatter(msg, dst, zeros_nh)
        a0, a1 = aggs[:N], aggs[N:]
        xpre, xs, xss, pool = _xpass_call(x, cp['Wx'], a0, a1, ug, seg2d)
        x_new = _xfin(xpre, x, xs, xss)
        u_new = _uupdate(u, pool, cp) if need_u else None
        return x_new, u_new

    cp = p['mol_convs'][0]
    gp_t, gd_t, ug = _proj_pair(x, u, cp)
    gp, g2 = _sc_gather_pair(gp_t, gd_t, srcm, dstm)
    epre1, msg, s1, ss1 = _epass1(bond_feats, gp, g2, cp,
                                  p['emb_bond_W'], p['emb_bond_b'])
    x, u = x_side(x, u, msg, ug, dstm, seg_mol, cp, True)

    cp = p['mol_convs'][1]
    gp_t, gd_t, ug = _proj_pair(x, u, cp)
    gp, g2 = _sc_gather_pair(gp_t, gd_t, srcm, dstm)
    e1, epre2, msg, s2, ss2 = _epass_mid2(epre1, bond_feats, p['emb_bond_W'],
                                          p['emb_bond_b'], s1, ss1, gp, g2, cp)
    x, _ = x_side(x, u, msg, ug, dstm, seg_mol, cp, False)

    u_rxn = _upool(x, seg_rxn)

    cp = p['rxn_convs'][0]
    gp_t, gd_t, ug = _proj_pair(x, u_rxn, cp)
    gp, g2 = _sc_gather_pair(gp_t, gd_t, srcr, dstr)
    e2, epre3, msg, s3, ss3 = _epass_mid(epre2, e1, s2, ss2, gp, g2, cp)
    x, _ = x_side(x, u_rxn, msg, ug, dstr, seg_rxn, cp, False)

    cp = p['rxn_convs'][1]
    gs_t, gd_t = _proj2(x, cp)
    g1, g2 = _sc_gather2(gs_t, gd_t, srcr, dstr)
    e3, epre4, s4, ss4 = _epass_last(epre3, e2, s3, ss3, g1, g2, cp)
    h = _efin_dec(epre4, e3, s4, ss4, p['dec_Ws'], p['dec_bs'])
    return h


# speedup vs baseline: 3.3517x; 1.0043x over previous
import functools

import jax
import jax.numpy as jnp
from jax import lax
from jax.experimental import pallas as pl
from jax.experimental.pallas import tpu as pltpu
from jax.experimental.pallas import tpu_sc as plsc

N = 10000
E = 160000
M = 500
R = 250
H = 128
NSEG = 512
NC, NS = 2, 16
NW = NC * NS
EPW = E // NW
CH = 128
NCH = EPW // CH
TAIL = EPW - NCH * CH
EPS = 1e-5

f32 = jnp.float32


def _mesh():
    return plsc.VectorSubcoreMesh(core_axis_name="c", subcore_axis_name="s",
                                  num_cores=NC, num_subcores=NS)


def _gather_pipelined(tables, idx_sel, n_out):
    ntab = len(tables)

    def body(srch, dsth, outs, isv, idv, bufs, bufsT, semg0, semg1, semw0, semw1):
        wid = lax.axis_index("s") * NC + lax.axis_index("c")
        base = wid * EPW
        pltpu.sync_copy(srch.at[pl.ds(base, EPW)], isv)
        pltpu.sync_copy(dsth.at[pl.ds(base, EPW)], idv)
        ivs = [isv, idv]

        def gstart(i, slot, semg):
            off = i * CH
            for t in range(ntab):
                pltpu.async_copy(
                    tables[t].at[ivs[idx_sel[t]].at[pl.ds(off, CH)]],
                    bufs[t].at[slot], semg)

        def gwait(slot, semg):
            for t in range(ntab):
                pltpu.make_async_copy(
                    tables[t].at[isv.at[pl.ds(0, CH)]], bufs[t].at[slot], semg
                ).wait()

        def wstart(i, slot, semw):
            off = base + i * CH
            for t in range(ntab):
                pltpu.async_copy(bufs[t].at[slot], outs[t].at[pl.ds(off, CH)], semw)

        def wwait(slot, semw):
            for t in range(ntab):
                pltpu.make_async_copy(
                    bufs[t].at[slot], outs[t].at[pl.ds(0, CH)], semw).wait()

        gstart(0, 0, semg0)
        gstart(1, 1, semg1)

        def step(i, carry):
            def for_slot(slot, semg, semw):
                gwait(slot, semg)
                wstart(i, slot, semw)
                wwait(slot, semw)

                @pl.when(i + 2 < NCH)
                def _():
                    gstart(i + 2, slot, semg)

            @pl.when(lax.rem(i, 2) == 0)
            def _():
                for_slot(0, semg0, semw0)

            @pl.when(lax.rem(i, 2) == 1)
            def _():
                for_slot(1, semg1, semw1)

            return carry

        lax.fori_loop(0, NCH, step, 0)

        offT = NCH * CH
        for t in range(ntab):
            pltpu.async_copy(
                tables[t].at[ivs[idx_sel[t]].at[pl.ds(offT, TAIL)]],
                bufsT[t], semg0)
        for t in range(ntab):
            pltpu.make_async_copy(
                tables[t].at[isv.at[pl.ds(0, TAIL)]], bufsT[t], semg0).wait()
        for t in range(ntab):
            pltpu.sync_copy(bufsT[t], outs[t].at[pl.ds(base + offT, TAIL)])

    return body


def _sc_gather_pair(t_pair, t_dst, src, dst):
    @functools.partial(
        pl.kernel,
        out_type=[jax.ShapeDtypeStruct((E, H), jnp.int32),
                  jax.ShapeDtypeStruct((E, H), f32)],
        mesh=_mesh(),
        scratch_types=[
            pltpu.VMEM((EPW,), jnp.int32), pltpu.VMEM((EPW,), jnp.int32),
            pltpu.VMEM((2, CH, H), jnp.int32), pltpu.VMEM((2, CH, H), f32),
            pltpu.VMEM((TAIL, H), jnp.int32), pltpu.VMEM((TAIL, H), f32),
            pltpu.SemaphoreType.DMA, pltpu.SemaphoreType.DMA,
            pltpu.SemaphoreType.DMA, pltpu.SemaphoreType.DMA,
        ],
    )
    def k(t1h, t2h, srch, dsth, gph, g2h,
          isv, idv, b1, b2, b1T, b2T, sg0, sg1, sw0, sw1):
        _gather_pipelined([t1h, t2h], [0, 1], 2)(
            srch, dsth, [gph, g2h], isv, idv,
            [b1, b2], [b1T, b2T], sg0, sg1, sw0, sw1)

    return k(t_pair, t_dst, src, dst)


def _sc_gather2(t_src, t_dst, src, dst):
    @functools.partial(
        pl.kernel,
        out_type=[jax.ShapeDtypeStruct((E, H), f32)] * 2,
        mesh=_mesh(),
        scratch_types=[
            pltpu.VMEM((EPW,), jnp.int32), pltpu.VMEM((EPW,), jnp.int32),
            pltpu.VMEM((2, CH, H), f32), pltpu.VMEM((2, CH, H), f32),
            pltpu.VMEM((TAIL, H), f32), pltpu.VMEM((TAIL, H), f32),
            pltpu.SemaphoreType.DMA, pltpu.SemaphoreType.DMA,
            pltpu.SemaphoreType.DMA, pltpu.SemaphoreType.DMA,
        ],
    )
    def k(t1h, t2h, srch, dsth, g1h, g2h,
          isv, idv, b1, b2, b1T, b2T, sg0, sg1, sw0, sw1):
        _gather_pipelined([t1h, t2h], [0, 1], 2)(
            srch, dsth, [g1h, g2h], isv, idv,
            [b1, b2], [b1T, b2T], sg0, sg1, sw0, sw1)

    return k(t_src, t_dst, src, dst)


def _sc_scatter(msg, dst, zeros_nh):
    @functools.partial(
        pl.kernel,
        out_type=jax.ShapeDtypeStruct((NC * N, H), f32),
        mesh=_mesh(),
        scratch_types=[
            pltpu.VMEM_SHARED((N, H), f32),
            pltpu.VMEM((CH,), jnp.int32), pltpu.VMEM((CH, H), f32),
            pltpu.VMEM((CH,), jnp.int32), pltpu.VMEM((CH, H), f32),
            pltpu.VMEM((TAIL,), jnp.int32), pltpu.VMEM((TAIL, H), f32),
            pltpu.SemaphoreType.DMA, pltpu.SemaphoreType.DMA,
            pltpu.SemaphoreType.DMA, pltpu.SemaphoreType.DMA,
        ],
    )
    def k(msgh, dsth, zh, outh, acc, i0, r0, i1, r1, idxT, rowsT,
          sm0, sm1, sa0, sa1):
        cid = lax.axis_index("c")
        sid = lax.axis_index("s")
        wid = sid * NC + cid
        base = wid * EPW

        @pl.when(sid < NS - 1)
        def _():
            pltpu.sync_copy(zh.at[pl.ds(sid * 640, 640)], acc.at[pl.ds(sid * 640, 640)])

        @pl.when(sid == NS - 1)
        def _():
            pltpu.sync_copy(zh.at[pl.ds(9600, 400)], acc.at[pl.ds(9600, 400)])

        plsc.subcore_barrier()

        def lstart(i, iv, rv, sem):
            off = base + i * CH
            pltpu.async_copy(msgh.at[pl.ds(off, CH)], rv, sem)
            pltpu.async_copy(dsth.at[pl.ds(off, CH)], iv, sem)

        def lwait(iv, rv, sem):
            pltpu.make_async_copy(msgh.at[pl.ds(0, CH)], rv, sem).wait()
            pltpu.make_async_copy(dsth.at[pl.ds(0, CH)], iv, sem).wait()

        lstart(0, i0, r0, sm0)

        def step(i, carry):
            def for_slot(iv, rv, sem, sad, iv2, rv2, sem2, sad2):
                lwait(iv, rv, sem)

                @pl.when(i >= 1)
                def _():
                    pltpu.make_async_copy(rv2, acc.at[iv2], sad2).wait()

                @pl.when(i + 1 < NCH)
                def _():
                    lstart(i + 1, iv2, rv2, sem2)

                pltpu.async_copy(rv, acc.at[iv], sad, add=True)

            @pl.when(lax.rem(i, 2) == 0)
            def _():
                for_slot(i0, r0, sm0, sa0, i1, r1, sm1, sa1)

            @pl.when(lax.rem(i, 2) == 1)
            def _():
                for_slot(i1, r1, sm1, sa1, i0, r0, sm0, sa0)

            return carry

        lax.fori_loop(0, NCH, step, 0)
        pltpu.make_async_copy(r0, acc.at[i0], sa0).wait()

        offT = base + NCH * CH
        pltpu.sync_copy(msgh.at[pl.ds(offT, TAIL)], rowsT)
        pltpu.sync_copy(dsth.at[pl.ds(offT, TAIL)], idxT)
        pltpu.sync_copy(rowsT, acc.at[idxT], add=True)

        plsc.subcore_barrier()

        @pl.when(sid < NS - 1)
        def _():
            pltpu.sync_copy(acc.at[pl.ds(sid * 640, 640)],
                            outh.at[pl.ds(cid * N + sid * 640, 640)])

        @pl.when(sid == NS - 1)
        def _():
            pltpu.sync_copy(acc.at[pl.ds(9600, 400)],
                            outh.at[pl.ds(cid * N + 9600, 400)])

    return k(msg, dst, zeros_nh)


def _dot(a, b):
    return jnp.dot(a, b, preferred_element_type=f32)


def _embed_xu_body(af, wa, ba, gf, wg, bg, xo, uo):
    xo[...] = _dot(af[...], wa[...]) + ba[...]
    uo[...] = _dot(gf[...], wg[...]) + bg[...]


def _embed_xu(atom_feats, wa, ba, gf_pad, wg, bg):
    return pl.pallas_call(
        _embed_xu_body,
        out_shape=[jax.ShapeDtypeStruct((N, H), f32),
                   jax.ShapeDtypeStruct((NSEG, H), f32)],
    )(atom_feats, wa, ba.reshape(1, H), gf_pad, wg, bg.reshape(1, H))


def _pack_pair(a, b):
    ai = lax.bitcast_convert_type(a, jnp.int32)
    bi = lax.bitcast_convert_type(b, jnp.int32)
    hi = (ai + jnp.int32(0x8000)) & jnp.int32(-65536)
    lo = lax.shift_right_logical(bi + jnp.int32(0x8000), 16)
    return hi | lo


def _unpack_pair(v):
    g1 = lax.bitcast_convert_type(v & jnp.int32(-65536), f32)
    g3 = lax.bitcast_convert_type(lax.shift_left(v, 16), f32)
    return g1, g3


def _proj_pair_body(x, ws, wd, wm, u, wug, bx, gpo, gdo, ugo):
    xv = x[...]
    a = _dot(xv, ws[...])
    b = _dot(xv, wm[...])
    gpo[...] = _pack_pair(a, b)
    gdo[...] = _dot(xv, wd[...])
    ugo[...] = _dot(u[...], wug[...]) + bx[...]


def _proj_pair(x, u, p):
    return pl.pallas_call(
        _proj_pair_body,
        out_shape=[jax.ShapeDtypeStruct((N, H), jnp.int32),
                   jax.ShapeDtypeStruct((N, H), f32),
                   jax.ShapeDtypeStruct((NSEG, H), f32)],
    )(x, p['Wsrc'], p['Wdst'], p['Wm'], u, p['Wug'], p['bx'].reshape(1, H))


def _proj2_body(x, ws, wd, gso, gdo):
    xv = x[...]
    gso[...] = _dot(xv, ws[...])
    gdo[...] = _dot(xv, wd[...])


def _proj2(x, p):
    return pl.pallas_call(
        _proj2_body,
        out_shape=[jax.ShapeDtypeStruct((N, H), f32)] * 2,
    )(x, p['Wsrc'], p['Wdst'])


def _acc_stats(i, epre, s_o, ss_o):
    s = jnp.sum(epre, axis=0, keepdims=True)
    ss = jnp.sum(epre * epre, axis=0, keepdims=True)

    @pl.when(i == 0)
    def _():
        s_o[...] = s
        ss_o[...] = ss

    @pl.when(i != 0)
    def _():
        s_o[...] += s
        ss_o[...] += ss


def _epass1_body(bf, wemb, bemb, gp, g2, we, be, epre_o, msg_o, s_o, ss_o):
    i = pl.program_id(0)
    e0 = _dot(bf[...], wemb[...]) + bemb[...]
    g1, xms = _unpack_pair(gp[...])
    epre = jnp.maximum(_dot(e0, we[...]) + g1 + g2[...] + be[...], 0.0)
    epre_o[...] = epre.astype(jnp.bfloat16)
    msg_o[...] = jax.nn.sigmoid(epre) * xms
    _acc_stats(i, epre, s_o, ss_o)


def _epass1(bond_feats, gp, g2, p, wemb, bemb):
    BE = 8000
    blk = lambda i: (i, 0)
    zero = lambda i: (0, 0)
    return pl.pallas_call(
        _epass1_body,
        grid=(E // BE,),
        in_specs=[pl.BlockSpec((BE, 64), blk), pl.BlockSpec((64, H), zero),
                  pl.BlockSpec((1, H), zero),
                  pl.BlockSpec((BE, H), blk), pl.BlockSpec((BE, H), blk),
                  pl.BlockSpec((H, H), zero), pl.BlockSpec((1, H), zero)],
        out_specs=[pl.BlockSpec((BE, H), blk), pl.BlockSpec((BE, H), blk),
                   pl.BlockSpec((1, H), zero), pl.BlockSpec((1, H), zero)],
        out_shape=[jax.ShapeDtypeStruct((E, H), jnp.bfloat16),
                   jax.ShapeDtypeStruct((E, H), f32),
                   jax.ShapeDtypeStruct((1, H), f32), jax.ShapeDtypeStruct((1, H), f32)],
    )(bond_feats, wemb, bemb.reshape(1, H), gp, g2, p['We'], p['be'].reshape(1, H))


def _epass_mid2_body(epre_p, bf, wemb, bemb, s_p, ss_p, gp, g2, we, be,
                     enew_o, epre_o, msg_o, s_o, ss_o):
    i = pl.program_id(0)
    mu = s_p[...] / E
    var = ss_p[...] / E - mu * mu
    e0 = _dot(bf[...], wemb[...]) + bemb[...]
    enew = (epre_p[...].astype(f32) - mu) * lax.rsqrt(var + EPS) + e0
    enew_o[...] = enew
    g1, xms = _unpack_pair(gp[...])
    epre = jnp.maximum(_dot(enew, we[...]) + g1 + g2[...] + be[...], 0.0)
    epre_o[...] = epre.astype(jnp.bfloat16)
    msg_o[...] = jax.nn.sigmoid(epre) * xms
    _acc_stats(i, epre, s_o, ss_o)


def _epass_mid2(epre_p, bond_feats, wemb, bemb, s_p, ss_p, gp, g2, p):
    BE = 8000
    blk = lambda i: (i, 0)
    zero = lambda i: (0, 0)
    return pl.pallas_call(
        _epass_mid2_body,
        grid=(E // BE,),
        in_specs=[pl.BlockSpec((BE, H), blk), pl.BlockSpec((BE, 64), blk),
                  pl.BlockSpec((64, H), zero), pl.BlockSpec((1, H), zero),
                  pl.BlockSpec((1, H), zero), pl.BlockSpec((1, H), zero),
                  pl.BlockSpec((BE, H), blk), pl.BlockSpec((BE, H), blk),
                  pl.BlockSpec((H, H), zero), pl.BlockSpec((1, H), zero)],
        out_specs=[pl.BlockSpec((BE, H), blk), pl.BlockSpec((BE, H), blk),
                   pl.BlockSpec((BE, H), blk),
                   pl.BlockSpec((1, H), zero), pl.BlockSpec((1, H), zero)],
        out_shape=[jax.ShapeDtypeStruct((E, H), f32),
                   jax.ShapeDtypeStruct((E, H), jnp.bfloat16),
                   jax.ShapeDtypeStruct((E, H), f32),
                   jax.ShapeDtypeStruct((1, H), f32), jax.ShapeDtypeStruct((1, H), f32)],
    )(epre_p, bond_feats, wemb, bemb.reshape(1, H), s_p, ss_p, gp, g2,
      p['We'], p['be'].reshape(1, H))


def _epass_mid_body(epre_p, eold, s_p, ss_p, gp, g2, we, be,
                    enew_o, epre_o, msg_o, s_o, ss_o):
    i = pl.program_id(0)
    mu = s_p[...] / E
    var = ss_p[...] / E - mu * mu
    enew = (epre_p[...].astype(f32) - mu) * lax.rsqrt(var + EPS) + eold[...]
    enew_o[...] = enew
    g1, xms = _unpack_pair(gp[...])
    epre = jnp.maximum(_dot(enew, we[...]) + g1 + g2[...] + be[...], 0.0)
    epre_o[...] = epre.astype(jnp.bfloat16)
    msg_o[...] = jax.nn.sigmoid(epre) * xms
    _acc_stats(i, epre, s_o, ss_o)


def _epass_mid(epre_p, eold, s_p, ss_p, gp, g2, p):
    BE = 8000
    blk = lambda i: (i, 0)
    zero = lambda i: (0, 0)
    return pl.pallas_call(
        _epass_mid_body,
        grid=(E // BE,),
        in_specs=[pl.BlockSpec((BE, H), blk), pl.BlockSpec((BE, H), blk),
                  pl.BlockSpec((1, H), zero), pl.BlockSpec((1, H), zero),
                  pl.BlockSpec((BE, H), blk), pl.BlockSpec((BE, H), blk),
                  pl.BlockSpec((H, H), zero), pl.BlockSpec((1, H), zero)],
        out_specs=[pl.BlockSpec((BE, H), blk), pl.BlockSpec((BE, H), blk),
                   pl.BlockSpec((BE, H), blk),
                   pl.BlockSpec((1, H), zero), pl.BlockSpec((1, H), zero)],
        out_shape=[jax.ShapeDtypeStruct((E, H), f32),
                   jax.ShapeDtypeStruct((E, H), jnp.bfloat16),
                   jax.ShapeDtypeStruct((E, H), f32),
                   jax.ShapeDtypeStruct((1, H), f32), jax.ShapeDtypeStruct((1, H), f32)],
    )(epre_p, eold, s_p, ss_p, gp, g2, p['We'], p['be'].reshape(1, H))


def _epass_last_body(epre_p, eold, s_p, ss_p, g1, g2, we, be,
                     enew_o, epre_o, s_o, ss_o):
    i = pl.program_id(0)
    mu = s_p[...] / E
    var = ss_p[...] / E - mu * mu
    enew = (epre_p[...].astype(f32) - mu) * lax.rsqrt(var + EPS) + eold[...]
    enew_o[...] = enew
    epre = jnp.maximum(_dot(enew, we[...]) + g1[...] + g2[...] + be[...], 0.0)
    epre_o[...] = epre.astype(jnp.bfloat16)
    _acc_stats(i, epre, s_o, ss_o)


def _epass_last(epre_p, eold, s_p, ss_p, g1, g2, p):
    BE = 8000
    blk = lambda i: (i, 0)
    zero = lambda i: (0, 0)
    return pl.pallas_call(
        _epass_last_body,
        grid=(E // BE,),
        in_specs=[pl.BlockSpec((BE, H), blk), pl.BlockSpec((BE, H), blk),
                  pl.BlockSpec((1, H), zero), pl.BlockSpec((1, H), zero),
                  pl.BlockSpec((BE, H), blk), pl.BlockSpec((BE, H), blk),
                  pl.BlockSpec((H, H), zero), pl.BlockSpec((1, H), zero)],
        out_specs=[pl.BlockSpec((BE, H), blk), pl.BlockSpec((BE, H), blk),
                   pl.BlockSpec((1, H), zero), pl.BlockSpec((1, H), zero)],
        out_shape=[jax.ShapeDtypeStruct((E, H), f32),
                   jax.ShapeDtypeStruct((E, H), jnp.bfloat16),
                   jax.ShapeDtypeStruct((1, H), f32), jax.ShapeDtypeStruct((1, H), f32)],
    )(epre_p, eold, s_p, ss_p, g1, g2, p['We'], p['be'].reshape(1, H))


def _xpass_body(bx_, x, wx, a0, a1, ug, seg, xpre_o, s_o, ss_o, pool_o):
    i = pl.program_id(0)
    segv = seg[...]
    oh = (segv == lax.broadcasted_iota(jnp.int32, (bx_, NSEG), 1)).astype(f32)
    ugs = _dot(oh, ug[...])
    xpre = jnp.maximum(_dot(x[...], wx[...]) + a0[...] + a1[...] + ugs, 0.0)
    xpre_o[...] = xpre
    x2 = jnp.concatenate([xpre, jnp.ones((bx_, H), f32)], axis=1)
    pool = lax.dot_general(oh, x2, (((0,), (0,)), ((), ())),
                           preferred_element_type=f32)
    s = jnp.sum(xpre, axis=0, keepdims=True)
    ss = jnp.sum(xpre * xpre, axis=0, keepdims=True)

    @pl.when(i == 0)
    def _():
        s_o[...] = s
        ss_o[...] = ss
        pool_o[...] = pool

    @pl.when(i != 0)
    def _():
        s_o[...] += s
        ss_o[...] += ss
        pool_o[...] += pool


def _xpass_call(x, wx, a0, a1, ug, seg2d):
    B = 2000
    blk = lambda i: (i, 0)
    zero = lambda i: (0, 0)
    return pl.pallas_call(
        functools.partial(_xpass_body, B),
        grid=(N // B,),
        in_specs=[pl.BlockSpec((B, H), blk), pl.BlockSpec((H, H), zero),
                  pl.BlockSpec((B, H), blk), pl.BlockSpec((B, H), blk),
                  pl.BlockSpec((NSEG, H), zero), pl.BlockSpec((B, 1), blk)],
        out_specs=[pl.BlockSpec((B, H), blk),
                   pl.BlockSpec((1, H), zero), pl.BlockSpec((1, H), zero),
                   pl.BlockSpec((NSEG, 2 * H), zero)],
        out_shape=[jax.ShapeDtypeStruct((N, H), f32),
                   jax.ShapeDtypeStruct((1, H), f32), jax.ShapeDtypeStruct((1, H), f32),
                   jax.ShapeDtypeStruct((NSEG, 2 * H), f32)],
    )(x, wx, a0, a1, ug, seg2d)


def _xfin_body(xpre, x, s, ss, o):
    mu = s[...] / N
    var = ss[...] / N - mu * mu
    o[...] = (xpre[...] - mu) * lax.rsqrt(var + EPS) + x[...]


def _xfin(xpre, x, s, ss):
    return pl.pallas_call(
        _xfin_body,
        out_shape=jax.ShapeDtypeStruct((N, H), f32),
    )(xpre, x, s, ss)


def _uupdate_body(u, wu, wg, bu, pool, uo):
    u_pool = pool[:, :H] / jnp.maximum(pool[:, H:], 1.0)
    upre = jnp.maximum(_dot(u[...], wu[...]) + _dot(u_pool, wg[...]) + bu[...], 0.0)
    mask = (lax.broadcasted_iota(jnp.int32, (NSEG, H), 0) < M).astype(f32)
    mu = jnp.sum(upre * mask, axis=0, keepdims=True) / M
    var = jnp.sum(((upre - mu) * mask) ** 2, axis=0, keepdims=True) / M
    uo[...] = (upre - mu) * lax.rsqrt(var + EPS) + u[...]


def _uupdate(u, pool, p):
    return pl.pallas_call(
        _uupdate_body,
        out_shape=jax.ShapeDtypeStruct((NSEG, H), f32),
    )(u, p['Wu'], p['Wg'], p['bu'].reshape(1, H), pool)


def _upool_body(x, seg, uo):
    segv = seg[...]
    oh = (segv == lax.broadcasted_iota(jnp.int32, (N, NSEG), 1)).astype(f32)
    x2 = jnp.concatenate([x[...], jnp.ones((N, H), f32)], axis=1)
    pool = lax.dot_general(oh, x2, (((0,), (0,)), ((), ())),
                           preferred_element_type=f32)
    uo[...] = pool[:, :H] / jnp.maximum(pool[:, H:], 1.0)


def _upool(x, seg2d):
    return pl.pallas_call(
        _upool_body,
        out_shape=jax.ShapeDtypeStruct((NSEG, H), f32),
    )(x, seg2d)


def _efin_dec_body(epre, e, s, ss, w1, b1, w2, b2, w3, b3, o):
    mu = s[...] / E
    var = ss[...] / E - mu * mu
    h = (epre[...].astype(f32) - mu) * lax.rsqrt(var + EPS) + e[...]
    h = jnp.maximum(_dot(h, w1[...]) + b1[...], 0.0)
    h = jnp.maximum(_dot(h, w2[...]) + b2[...], 0.0)
    o[...] = _dot(h, w3[...]) + b3[...]


def _efin_dec(epre, e, s, ss, dec_ws, dec_bs):
    BE = 8000
    blk = lambda i: (i, 0)
    zero = lambda i: (0, 0)
    w1, w2, w3 = dec_ws
    b1, b2, b3 = dec_bs
    return pl.pallas_call(
        _efin_dec_body,
        grid=(E // BE,),
        in_specs=[pl.BlockSpec((BE, H), blk), pl.BlockSpec((BE, H), blk),
                  pl.BlockSpec((1, H), zero), pl.BlockSpec((1, H), zero),
                  pl.BlockSpec((H, H), zero), pl.BlockSpec((1, H), zero),
                  pl.BlockSpec((H, 64), zero), pl.BlockSpec((1, 64), zero),
                  pl.BlockSpec((64, 3), zero), pl.BlockSpec((1, 3), zero)],
        out_specs=pl.BlockSpec((BE, 3), blk),
        out_shape=jax.ShapeDtypeStruct((E, 3), f32),
    )(epre, e, s, ss, w1, b1.reshape(1, H), w2, b2.reshape(1, 64), w3,
      b3.reshape(1, 3))


def kernel(atom_feats, bond_feats, global_feats, mol_edge_index, rxn_edge_index,
           atom2mol, atom2rxn, params):
    p = params
    srcm = mol_edge_index[0].astype(jnp.int32)
    dstm = mol_edge_index[1].astype(jnp.int32)
    srcr = rxn_edge_index[0].astype(jnp.int32)
    dstr = rxn_edge_index[1].astype(jnp.int32)
    seg_mol = atom2mol.astype(jnp.int32).reshape(N, 1)
    seg_rxn = atom2rxn.astype(jnp.int32).reshape(N, 1)
    zeros_nh = jnp.zeros((N, H), f32)

    gf_pad = jnp.zeros((NSEG, global_feats.shape[1]), f32).at[:M].set(global_feats)

    x, u = _embed_xu(atom_feats, p['emb_atom_W'], p['emb_atom_b'],
                     gf_pad, p['emb_glob_W'], p['emb_glob_b'])

    def x_side(x, u, msg, ug, dst, seg2d, cp, need_u):
        aggs = _sc_scatter(msg, dst, zeros_nh)
        a0, a1 = aggs[:N], aggs[N:]
        xpre, xs, xss, pool = _xpass_call(x, cp['Wx'], a0, a1, ug, seg2d)
        x_new = _xfin(xpre, x, xs, xss)
        u_new = _uupdate(u, pool, cp) if need_u else None
        return x_new, u_new

    cp = p['mol_convs'][0]
    gp_t, gd_t, ug = _proj_pair(x, u, cp)
    gp, g2 = _sc_gather_pair(gp_t, gd_t, srcm, dstm)
    epre1, msg, s1, ss1 = _epass1(bond_feats, gp, g2, cp,
                                  p['emb_bond_W'], p['emb_bond_b'])
    x, u = x_side(x, u, msg, ug, dstm, seg_mol, cp, True)

    cp = p['mol_convs'][1]
    gp_t, gd_t, ug = _proj_pair(x, u, cp)
    gp, g2 = _sc_gather_pair(gp_t, gd_t, srcm, dstm)
    e1, epre2, msg, s2, ss2 = _epass_mid2(epre1, bond_feats, p['emb_bond_W'],
                                          p['emb_bond_b'], s1, ss1, gp, g2, cp)
    x, _ = x_side(x, u, msg, ug, dstm, seg_mol, cp, False)

    u_rxn = _upool(x, seg_rxn)

    cp = p['rxn_convs'][0]
    gp_t, gd_t, ug = _proj_pair(x, u_rxn, cp)
    gp, g2 = _sc_gather_pair(gp_t, gd_t, srcr, dstr)
    e2, epre3, msg, s3, ss3 = _epass_mid(epre2, e1, s2, ss2, gp, g2, cp)
    x, _ = x_side(x, u_rxn, msg, ug, dstr, seg_rxn, cp, False)

    cp = p['rxn_convs'][1]
    gs_t, gd_t = _proj2(x, cp)
    g1, g2 = _sc_gather2(gs_t, gd_t, srcr, dstr)
    e3, epre4, s4, ss4 = _epass_last(epre3, e2, s3, ss3, g1, g2, cp)
    h = _efin_dec(epre4, e3, s4, ss4, p['dec_Ws'], p['dec_bs'])
    return h
